# Initial kernel scaffold; baseline (speedup 1.0000x reference)
import jax
import jax.numpy as jnp
from jax.experimental import pallas as pl


def kernel(vectors, h, edge_index, edge_fea, in_w1, in_b1, in_w2, in_b2, out_w1, out_b1, out_w2, out_b2, node_w1, node_b1, node_w2, node_b2):
    raise NotImplementedError("write your pallas kernel here")


# speedup vs baseline: 1.3784x; 1.3784x over previous
import functools
import numpy as np
import jax
import jax.numpy as jnp
from jax import lax
from jax.experimental import pallas as pl
from jax.experimental.pallas import tpu as pltpu

N = 10000
E = 320000
V = 3
H = 128

NP = 10240
NW = 32
C = 128
EP = 323584
EPW = EP // NW
NCH = EPW // C
EBLK = 512
NBLK = 512

_INTERPRET = False


def _silu(x):
    return x * jax.nn.sigmoid(x)


def _sel_matrices():
    S1 = np.zeros((16, 32), np.float32)
    S2 = np.zeros((16, 32), np.float32)
    S3 = np.zeros((32, 16), np.float32)
    T1 = np.zeros((16, 32), np.float32)
    T2 = np.zeros((16, 32), np.float32)
    T3 = np.zeros((32, 16), np.float32)
    for i in range(3):
        for j in range(3):
            for k in range(3):
                m = i * 9 + j * 3 + k
                S1[j * 3 + i, m] = 1.0
                S2[j * 3 + k, m] = 1.0
                S3[m, i * 3 + k] = 1.0
                T1[i * 3 + j, m] = 1.0
                T2[j * 3 + k, m] = 1.0
                T3[m, i * 3 + k] = 1.0
    return S1, S2, S3, T1, T2, T3


_S1, _S2, _S3, _T1, _T2, _T3 = _sel_matrices()


def _k0_body(h_ref, wr_ref, wcb_ref, b1_ref, a_ref, b_ref):
    h = h_ref[...]
    a_ref[...] = jnp.dot(h, wr_ref[...], preferred_element_type=jnp.float32)
    b_ref[...] = jnp.dot(h, wcb_ref[...], preferred_element_type=jnp.float32) + b1_ref[...]


def _k0(hp, wr, wc, b1):
    grid = NP // NBLK
    return pl.pallas_call(
        _k0_body,
        grid=(grid,),
        in_specs=[
            pl.BlockSpec((NBLK, H), lambda i: (i, 0)),
            pl.BlockSpec((H, H), lambda i: (0, 0)),
            pl.BlockSpec((H, H), lambda i: (0, 0)),
            pl.BlockSpec((1, H), lambda i: (0, 0)),
        ],
        out_specs=[
            pl.BlockSpec((NBLK, H), lambda i: (i, 0)),
            pl.BlockSpec((NBLK, H), lambda i: (i, 0)),
        ],
        out_shape=[
            jax.ShapeDtypeStruct((NP, H), jnp.float32),
            jax.ShapeDtypeStruct((NP, H), jnp.float32),
        ],
        interpret=_INTERPRET,
    )(hp, wr, wc, b1)


def _k2_body(ag_ref, bg_ref, zi_ref, zj_ref, fea_ref,
             wsf_ref, w2_ref, b2_ref, ow1_ref, ob1_ref, ow2_ref, ob2_ref,
             s1_ref, s2_ref, s3_ref, t1_ref, t2_ref, t3_ref,
             msg_ref, tail_ref):
    zi = zi_ref[...]
    zj = zj_ref[...]
    t = (jnp.dot(zj, s1_ref[...], preferred_element_type=jnp.float32)
         * jnp.dot(zi, s2_ref[...], preferred_element_type=jnp.float32))
    scal = jnp.dot(t, s3_ref[...], preferred_element_type=jnp.float32)
    ss = jnp.sum(scal * scal, axis=1, keepdims=True)
    nrm = jnp.sqrt(ss)
    scal = scal * (1.0 / jnp.maximum(nrm, 1e-12))

    u = jnp.concatenate([scal, fea_ref[...]], axis=1)
    x = _silu(ag_ref[...] + bg_ref[...]
              + jnp.dot(u, wsf_ref[...], preferred_element_type=jnp.float32))
    msg = _silu(jnp.dot(x, w2_ref[...], preferred_element_type=jnp.float32) + b2_ref[...])

    y = _silu(jnp.dot(msg, ow1_ref[...], preferred_element_type=jnp.float32) + ob1_ref[...])
    vs = jnp.dot(y, ow2_ref[...], preferred_element_type=jnp.float32) + ob2_ref[...]

    t2 = (jnp.dot(zj, t1_ref[...], preferred_element_type=jnp.float32)
          * jnp.dot(vs, t2_ref[...], preferred_element_type=jnp.float32))
    vec = jnp.dot(t2, t3_ref[...], preferred_element_type=jnp.float32)

    eid = pl.program_id(0) * EBLK + lax.broadcasted_iota(jnp.int32, (EBLK, 1), 0)
    maskf = jnp.where(eid < E, 1.0, 0.0)

    msg_ref[...] = msg * maskf
    tail = jnp.concatenate(
        [vec[:, :9], jnp.ones((EBLK, 1), jnp.float32),
         jnp.zeros((EBLK, 6), jnp.float32)], axis=1)
    tail_ref[...] = tail * maskf


def _k2(ag, bg, zi, zj, fea, wsf, w2, b2, ow1, ob1, ow2, ob2):
    grid = EP // EBLK
    full = lambda i: (0, 0)
    blk = lambda i: (i, 0)
    return pl.pallas_call(
        _k2_body,
        grid=(grid,),
        in_specs=[
            pl.BlockSpec((EBLK, H), blk),
            pl.BlockSpec((EBLK, H), blk),
            pl.BlockSpec((EBLK, 16), blk),
            pl.BlockSpec((EBLK, 16), blk),
            pl.BlockSpec((EBLK, 16), blk),
            pl.BlockSpec((32, H), full),
            pl.BlockSpec((H, H), full),
            pl.BlockSpec((1, H), full),
            pl.BlockSpec((H, H), full),
            pl.BlockSpec((1, H), full),
            pl.BlockSpec((H, 16), full),
            pl.BlockSpec((1, 16), full),
            pl.BlockSpec((16, 32), full),
            pl.BlockSpec((16, 32), full),
            pl.BlockSpec((32, 16), full),
            pl.BlockSpec((16, 32), full),
            pl.BlockSpec((16, 32), full),
            pl.BlockSpec((32, 16), full),
        ],
        out_specs=[
            pl.BlockSpec((EBLK, H), blk),
            pl.BlockSpec((EBLK, 16), blk),
        ],
        out_shape=[
            jax.ShapeDtypeStruct((EP, H), jnp.float32),
            jax.ShapeDtypeStruct((EP, 16), jnp.float32),
        ],
        interpret=_INTERPRET,
    )(ag, bg, zi, zj, fea, wsf, w2, b2, ow1, ob1, ow2, ob2,
      jnp.asarray(_S1), jnp.asarray(_S2), jnp.asarray(_S3),
      jnp.asarray(_T1), jnp.asarray(_T2), jnp.asarray(_T3))


def _k4_body(h_ref, accm_ref, accv_ref, vecp_ref,
             wn1a_ref, wn1b_ref, nb1_ref, wn2_ref, nb2_ref,
             hnew_ref, vout_ref):
    h = h_ref[...]
    tot = accm_ref[0] + accm_ref[1]
    vsum = accv_ref[0] + accv_ref[1]
    cnt = vsum[:, 9:10]
    recip = 1.0 / jnp.maximum(cnt, 1.0)
    vout_ref[...] = vsum * recip + vecp_ref[...]
    z = _silu(jnp.dot(h, wn1a_ref[...], preferred_element_type=jnp.float32)
              + jnp.dot(tot, wn1b_ref[...], preferred_element_type=jnp.float32)
              + nb1_ref[...])
    hnew_ref[...] = jnp.dot(z, wn2_ref[...], preferred_element_type=jnp.float32) + nb2_ref[...] + h


def _k4(hp, accm, accv, vecp, wn1a, wn1b, nb1, wn2, nb2):
    grid = NP // NBLK
    full = lambda i: (0, 0)
    return pl.pallas_call(
        _k4_body,
        grid=(grid,),
        in_specs=[
            pl.BlockSpec((NBLK, H), lambda i: (i, 0)),
            pl.BlockSpec((2, NBLK, H), lambda i: (0, i, 0)),
            pl.BlockSpec((2, NBLK, 16), lambda i: (0, i, 0)),
            pl.BlockSpec((NBLK, 16), lambda i: (i, 0)),
            pl.BlockSpec((H, H), full),
            pl.BlockSpec((H, H), full),
            pl.BlockSpec((1, H), full),
            pl.BlockSpec((H, H), full),
            pl.BlockSpec((1, H), full),
        ],
        out_specs=[
            pl.BlockSpec((NBLK, H), lambda i: (i, 0)),
            pl.BlockSpec((NBLK, 16), lambda i: (i, 0)),
        ],
        out_shape=[
            jax.ShapeDtypeStruct((NP, H), jnp.float32),
            jax.ShapeDtypeStruct((NP, 16), jnp.float32),
        ],
        interpret=_INTERPRET,
    )(hp, accm, accv, vecp, wn1a, wn1b, nb1, wn2, nb2)


def kernel(vectors, h, edge_index, edge_fea,
           in_w1, in_b1, in_w2, in_b2,
           out_w1, out_b1, out_w2, out_b2,
           node_w1, node_b1, node_w2, node_b2):
    f32 = jnp.float32
    row = edge_index[0].astype(jnp.int32)
    col = edge_index[1].astype(jnp.int32)
    rowp = jnp.pad(row, (0, EP - E))
    colp = jnp.pad(col, (0, EP - E))
    feap = jnp.pad(edge_fea.astype(f32), ((0, EP - E), (0, 16 - 4)))

    hp = jnp.pad(h.astype(f32), ((0, NP - N), (0, 0)))
    vecp = jnp.pad(vectors.astype(f32).reshape(N, V * V), ((0, NP - N), (0, 16 - V * V)))

    wr = in_w1[9:137]
    wc = in_w1[137:265]
    wsf = jnp.zeros((32, H), f32).at[0:9].set(in_w1[0:9]).at[16:20].set(in_w1[265:269])
    b1 = in_b1.reshape(1, H)

    A, B = _k0(hp, wr, wc, b1)

    ag = A[rowp]
    bg = B[colp]
    zi = vecp[rowp]
    zj = vecp[colp]

    ow2p = jnp.zeros((H, 16), f32).at[:, :9].set(out_w2)
    ob2p = jnp.zeros((1, 16), f32).at[0, :9].set(out_b2)

    msg, tail = _k2(ag, bg, zi, zj, feap,
                    wsf, in_w2, in_b2.reshape(1, H),
                    out_w1, out_b1.reshape(1, H), ow2p, ob2p)

    accm0 = jax.ops.segment_sum(msg, rowp, num_segments=NP)
    accv0 = jax.ops.segment_sum(tail, rowp, num_segments=NP)
    accm = jnp.stack([accm0, jnp.zeros_like(accm0)])
    accv = jnp.stack([accv0, jnp.zeros_like(accv0)])

    wn1a = node_w1[:H]
    wn1b = node_w1[H:]

    hnew, vout = _k4(hp, accm, accv, vecp,
                     wn1a, wn1b, node_b1.reshape(1, H),
                     node_w2, node_b2.reshape(1, H))

    return (vout[:N, :9].reshape(N, 3, V), hnew[:N])


# speedup vs baseline: 3.6196x; 2.6259x over previous
import functools
import numpy as np
import jax
import jax.numpy as jnp
from jax import lax
from jax.experimental import pallas as pl
from jax.experimental.pallas import tpu as pltpu
from jax.experimental.pallas import tpu_sc as plsc

N = 10000
E = 320000
V = 3
H = 128

NP = 10240
NW = 32
C = 128
EP = 327680
EPW = EP // NW
NCH = EPW // C
EBLK = 512
NBLK = 512

_INTERPRET = False


def _silu(x):
    return x * jax.nn.sigmoid(x)


def _sel_matrices():
    S1 = np.zeros((16, 32), np.float32)
    S2 = np.zeros((16, 32), np.float32)
    S3 = np.zeros((32, 16), np.float32)
    T1 = np.zeros((16, 32), np.float32)
    T2 = np.zeros((16, 32), np.float32)
    T3 = np.zeros((32, 16), np.float32)
    for i in range(3):
        for j in range(3):
            for k in range(3):
                m = i * 9 + j * 3 + k
                S1[j * 3 + i, m] = 1.0
                S2[j * 3 + k, m] = 1.0
                S3[m, i * 3 + k] = 1.0
                T1[i * 3 + j, m] = 1.0
                T2[j * 3 + k, m] = 1.0
                T3[m, i * 3 + k] = 1.0
    return S1, S2, S3, T1, T2, T3


_S1, _S2, _S3, _T1, _T2, _T3 = _sel_matrices()


def _k0_body(h_ref, wr_ref, wcb_ref, b1_ref, a_ref, b_ref):
    h = h_ref[...]
    a_ref[...] = jnp.dot(h, wr_ref[...], preferred_element_type=jnp.float32)
    b_ref[...] = jnp.dot(h, wcb_ref[...], preferred_element_type=jnp.float32) + b1_ref[...]


def _k0(hp, wr, wc, b1):
    grid = NP // NBLK
    return pl.pallas_call(
        _k0_body,
        grid=(grid,),
        in_specs=[
            pl.BlockSpec((NBLK, H), lambda i: (i, 0)),
            pl.BlockSpec((H, H), lambda i: (0, 0)),
            pl.BlockSpec((H, H), lambda i: (0, 0)),
            pl.BlockSpec((1, H), lambda i: (0, 0)),
        ],
        out_specs=[
            pl.BlockSpec((NBLK, H), lambda i: (i, 0)),
            pl.BlockSpec((NBLK, H), lambda i: (i, 0)),
        ],
        out_shape=[
            jax.ShapeDtypeStruct((NP, H), jnp.float32),
            jax.ShapeDtypeStruct((NP, H), jnp.float32),
        ],
        interpret=_INTERPRET,
    )(hp, wr, wc, b1)


def _k1_body(row3, col3, a_hbm, b_hbm, vec_hbm,
             ag_o, bg_o, zi_o, zj_o,
             idxr, idxc, abuf, bbuf, zibuf, zjbuf,
             gsem0, gsem1, wsem):
    wid = lax.axis_index("s") * 2 + lax.axis_index("c")
    base = wid * EPW
    pltpu.sync_copy(row3.at[wid], idxr)
    pltpu.sync_copy(col3.at[wid], idxc)

    def gather(j, b, sem):
        return [
            pltpu.async_copy(a_hbm.at[idxr.at[j]], abuf.at[b], sem),
            pltpu.async_copy(b_hbm.at[idxc.at[j]], bbuf.at[b], sem),
            pltpu.async_copy(vec_hbm.at[idxr.at[j]], zibuf.at[b], sem),
            pltpu.async_copy(vec_hbm.at[idxc.at[j]], zjbuf.at[b], sem),
        ]

    def writeback(j, b):
        off = base + j * C
        return [
            pltpu.async_copy(abuf.at[b], ag_o.at[pl.ds(off, C)], wsem),
            pltpu.async_copy(bbuf.at[b], bg_o.at[pl.ds(off, C)], wsem),
            pltpu.async_copy(zibuf.at[b], zi_o.at[pl.ds(off, C)], wsem),
            pltpu.async_copy(zjbuf.at[b], zj_o.at[pl.ds(off, C)], wsem),
        ]

    def pair(k, carry):
        j0 = 2 * k
        j1 = 2 * k + 1
        g0 = gather(j0, 0, gsem0)
        g1 = gather(j1, 1, gsem1)
        for cp in g0:
            cp.wait()
        w0 = writeback(j0, 0)
        for cp in g1:
            cp.wait()
        w1 = writeback(j1, 1)
        for cp in w0 + w1:
            cp.wait()
        return carry

    lax.fori_loop(0, NCH // 2, pair, 0)


def _k1(row3, col3, A, B, vecp):
    f32 = jnp.float32
    mesh = plsc.VectorSubcoreMesh(core_axis_name="c", subcore_axis_name="s")
    fn = functools.partial(
        pl.kernel,
        out_type=[
            jax.ShapeDtypeStruct((EP, H), f32),
            jax.ShapeDtypeStruct((EP, H), f32),
            jax.ShapeDtypeStruct((EP, 16), f32),
            jax.ShapeDtypeStruct((EP, 16), f32),
        ],
        mesh=mesh,
        scratch_types=[
            pltpu.VMEM((NCH, C), jnp.int32),
            pltpu.VMEM((NCH, C), jnp.int32),
            pltpu.VMEM((2, C, H), f32),
            pltpu.VMEM((2, C, H), f32),
            pltpu.VMEM((2, C, 16), f32),
            pltpu.VMEM((2, C, 16), f32),
            pltpu.SemaphoreType.DMA,
            pltpu.SemaphoreType.DMA,
            pltpu.SemaphoreType.DMA,
        ],
        compiler_params=pltpu.CompilerParams(use_tc_tiling_on_sc=False),
    )(_k1_body)
    return fn(row3, col3, A, B, vecp)


NPS = NP // 16
ZR = 64


def _k3_body(row3, msg_hbm, tail_hbm, accm_o, accv_o,
             idx, mbuf, vbuf, zm, zv, accm_sh, accv_sh):
    cid = lax.axis_index("c")
    sid = lax.axis_index("s")
    wid = sid * 2 + cid
    base = wid * EPW
    pltpu.sync_copy(row3.at[wid], idx)

    def zrow(i, carry):
        for l in range(H // 16):
            zm[i, pl.ds(l * 16, 16)] = jnp.zeros((16,), jnp.float32)
        zv[i, :] = jnp.zeros((16,), jnp.float32)
        return carry

    lax.fori_loop(0, ZR, zrow, 0)

    def zcp(k, carry):
        r = sid * NPS + k * ZR
        pltpu.sync_copy(zm, accm_sh.at[pl.ds(r, ZR)])
        pltpu.sync_copy(zv, accv_sh.at[pl.ds(r, ZR)])
        return carry

    lax.fori_loop(0, NPS // ZR, zcp, 0)
    plsc.subcore_barrier()

    def chunk(j, carry):
        off = base + j * C
        pltpu.sync_copy(msg_hbm.at[pl.ds(off, C)], mbuf)
        pltpu.sync_copy(tail_hbm.at[pl.ds(off, C)], vbuf)
        pltpu.sync_copy(mbuf, accm_sh.at[idx.at[j]], add=True)
        pltpu.sync_copy(vbuf, accv_sh.at[idx.at[j]], add=True)
        return carry

    lax.fori_loop(0, NCH, chunk, 0)
    plsc.subcore_barrier()

    r = sid * NPS
    pltpu.sync_copy(accm_sh.at[pl.ds(r, NPS)], accm_o.at[cid, pl.ds(r, NPS)])
    pltpu.sync_copy(accv_sh.at[pl.ds(r, NPS)], accv_o.at[cid, pl.ds(r, NPS)])


def _k3(row3, msg, tail):
    f32 = jnp.float32
    mesh = plsc.VectorSubcoreMesh(core_axis_name="c", subcore_axis_name="s")
    fn = functools.partial(
        pl.kernel,
        out_type=[
            jax.ShapeDtypeStruct((2, NP, H), f32),
            jax.ShapeDtypeStruct((2, NP, 16), f32),
        ],
        mesh=mesh,
        scratch_types=[
            pltpu.VMEM((NCH, C), jnp.int32),
            pltpu.VMEM((C, H), f32),
            pltpu.VMEM((C, 16), f32),
            pltpu.VMEM((ZR, H), f32),
            pltpu.VMEM((ZR, 16), f32),
            pltpu.VMEM_SHARED((NP, H), f32),
            pltpu.VMEM_SHARED((NP, 16), f32),
        ],
        compiler_params=pltpu.CompilerParams(use_tc_tiling_on_sc=False),
    )(_k3_body)
    return fn(row3, msg, tail)


def _k2_body(ag_ref, bg_ref, zi_ref, zj_ref, fea_ref,
             wsf_ref, w2_ref, b2_ref, ow1_ref, ob1_ref, ow2_ref, ob2_ref,
             s1_ref, s2_ref, s3_ref, t1_ref, t2_ref, t3_ref,
             msg_ref, tail_ref):
    zi = zi_ref[...]
    zj = zj_ref[...]
    t = (jnp.dot(zj, s1_ref[...], preferred_element_type=jnp.float32)
         * jnp.dot(zi, s2_ref[...], preferred_element_type=jnp.float32))
    scal = jnp.dot(t, s3_ref[...], preferred_element_type=jnp.float32)
    ss = jnp.sum(scal * scal, axis=1, keepdims=True)
    nrm = jnp.sqrt(ss)
    scal = scal * (1.0 / jnp.maximum(nrm, 1e-12))

    u = jnp.concatenate([scal, fea_ref[...]], axis=1)
    x = _silu(ag_ref[...] + bg_ref[...]
              + jnp.dot(u, wsf_ref[...], preferred_element_type=jnp.float32))
    msg = _silu(jnp.dot(x, w2_ref[...], preferred_element_type=jnp.float32) + b2_ref[...])

    y = _silu(jnp.dot(msg, ow1_ref[...], preferred_element_type=jnp.float32) + ob1_ref[...])
    vs = jnp.dot(y, ow2_ref[...], preferred_element_type=jnp.float32) + ob2_ref[...]

    t2 = (jnp.dot(zj, t1_ref[...], preferred_element_type=jnp.float32)
          * jnp.dot(vs, t2_ref[...], preferred_element_type=jnp.float32))
    vec = jnp.dot(t2, t3_ref[...], preferred_element_type=jnp.float32)

    eid = pl.program_id(0) * EBLK + lax.broadcasted_iota(jnp.int32, (EBLK, 1), 0)
    maskf = jnp.where(eid < E, 1.0, 0.0)

    msg_ref[...] = msg * maskf
    tail = jnp.concatenate(
        [vec[:, :9], jnp.ones((EBLK, 1), jnp.float32),
         jnp.zeros((EBLK, 6), jnp.float32)], axis=1)
    tail_ref[...] = tail * maskf


def _k2(ag, bg, zi, zj, fea, wsf, w2, b2, ow1, ob1, ow2, ob2):
    grid = EP // EBLK
    full = lambda i: (0, 0)
    blk = lambda i: (i, 0)
    return pl.pallas_call(
        _k2_body,
        grid=(grid,),
        in_specs=[
            pl.BlockSpec((EBLK, H), blk),
            pl.BlockSpec((EBLK, H), blk),
            pl.BlockSpec((EBLK, 16), blk),
            pl.BlockSpec((EBLK, 16), blk),
            pl.BlockSpec((EBLK, 16), blk),
            pl.BlockSpec((32, H), full),
            pl.BlockSpec((H, H), full),
            pl.BlockSpec((1, H), full),
            pl.BlockSpec((H, H), full),
            pl.BlockSpec((1, H), full),
            pl.BlockSpec((H, 16), full),
            pl.BlockSpec((1, 16), full),
            pl.BlockSpec((16, 32), full),
            pl.BlockSpec((16, 32), full),
            pl.BlockSpec((32, 16), full),
            pl.BlockSpec((16, 32), full),
            pl.BlockSpec((16, 32), full),
            pl.BlockSpec((32, 16), full),
        ],
        out_specs=[
            pl.BlockSpec((EBLK, H), blk),
            pl.BlockSpec((EBLK, 16), blk),
        ],
        out_shape=[
            jax.ShapeDtypeStruct((EP, H), jnp.float32),
            jax.ShapeDtypeStruct((EP, 16), jnp.float32),
        ],
        interpret=_INTERPRET,
    )(ag, bg, zi, zj, fea, wsf, w2, b2, ow1, ob1, ow2, ob2,
      jnp.asarray(_S1), jnp.asarray(_S2), jnp.asarray(_S3),
      jnp.asarray(_T1), jnp.asarray(_T2), jnp.asarray(_T3))


def _k4_body(h_ref, accm_ref, accv_ref, vecp_ref,
             wn1a_ref, wn1b_ref, nb1_ref, wn2_ref, nb2_ref,
             hnew_ref, vout_ref):
    h = h_ref[...]
    tot = accm_ref[0] + accm_ref[1]
    vsum = accv_ref[0] + accv_ref[1]
    cnt = vsum[:, 9:10]
    recip = 1.0 / jnp.maximum(cnt, 1.0)
    vout_ref[...] = vsum * recip + vecp_ref[...]
    z = _silu(jnp.dot(h, wn1a_ref[...], preferred_element_type=jnp.float32)
              + jnp.dot(tot, wn1b_ref[...], preferred_element_type=jnp.float32)
              + nb1_ref[...])
    hnew_ref[...] = jnp.dot(z, wn2_ref[...], preferred_element_type=jnp.float32) + nb2_ref[...] + h


def _k4(hp, accm, accv, vecp, wn1a, wn1b, nb1, wn2, nb2):
    grid = NP // NBLK
    full = lambda i: (0, 0)
    return pl.pallas_call(
        _k4_body,
        grid=(grid,),
        in_specs=[
            pl.BlockSpec((NBLK, H), lambda i: (i, 0)),
            pl.BlockSpec((2, NBLK, H), lambda i: (0, i, 0)),
            pl.BlockSpec((2, NBLK, 16), lambda i: (0, i, 0)),
            pl.BlockSpec((NBLK, 16), lambda i: (i, 0)),
            pl.BlockSpec((H, H), full),
            pl.BlockSpec((H, H), full),
            pl.BlockSpec((1, H), full),
            pl.BlockSpec((H, H), full),
            pl.BlockSpec((1, H), full),
        ],
        out_specs=[
            pl.BlockSpec((NBLK, H), lambda i: (i, 0)),
            pl.BlockSpec((NBLK, 16), lambda i: (i, 0)),
        ],
        out_shape=[
            jax.ShapeDtypeStruct((NP, H), jnp.float32),
            jax.ShapeDtypeStruct((NP, 16), jnp.float32),
        ],
        interpret=_INTERPRET,
    )(hp, accm, accv, vecp, wn1a, wn1b, nb1, wn2, nb2)


def kernel(vectors, h, edge_index, edge_fea,
           in_w1, in_b1, in_w2, in_b2,
           out_w1, out_b1, out_w2, out_b2,
           node_w1, node_b1, node_w2, node_b2):
    f32 = jnp.float32
    row = edge_index[0].astype(jnp.int32)
    col = edge_index[1].astype(jnp.int32)
    rowp = jnp.pad(row, (0, EP - E))
    colp = jnp.pad(col, (0, EP - E))
    feap = jnp.pad(edge_fea.astype(f32), ((0, EP - E), (0, 16 - 4)))

    hp = jnp.pad(h.astype(f32), ((0, NP - N), (0, 0)))
    vecp = jnp.pad(vectors.astype(f32).reshape(N, V * V), ((0, NP - N), (0, 16 - V * V)))

    wr = in_w1[9:137]
    wc = in_w1[137:265]
    wsf = jnp.zeros((32, H), f32).at[0:9].set(in_w1[0:9]).at[16:20].set(in_w1[265:269])
    b1 = in_b1.reshape(1, H)

    A, B = _k0(hp, wr, wc, b1)

    row3 = rowp.reshape(NW, NCH, C)
    col3 = colp.reshape(NW, NCH, C)
    ag, bg, zi, zj = _k1(row3, col3, A, B, vecp)

    ow2p = jnp.zeros((H, 16), f32).at[:, :9].set(out_w2)
    ob2p = jnp.zeros((1, 16), f32).at[0, :9].set(out_b2)

    msg, tail = _k2(ag, bg, zi, zj, feap,
                    wsf, in_w2, in_b2.reshape(1, H),
                    out_w1, out_b1.reshape(1, H), ow2p, ob2p)

    accm, accv = _k3(row3, msg, tail)

    wn1a = node_w1[:H]
    wn1b = node_w1[H:]

    hnew, vout = _k4(hp, accm, accv, vecp,
                     wn1a, wn1b, node_b1.reshape(1, H),
                     node_w2, node_b2.reshape(1, H))

    return (vout[:N, :9].reshape(N, 3, V), hnew[:N])


# speedup vs baseline: 3.7996x; 1.0497x over previous
import functools
import numpy as np
import jax
import jax.numpy as jnp
from jax import lax
from jax.experimental import pallas as pl
from jax.experimental.pallas import tpu as pltpu
from jax.experimental.pallas import tpu_sc as plsc

N = 10000
E = 320000
V = 3
H = 128

NP = 10240
NW = 32
C = 128
EP = 327680
EPW = EP // NW
NCH = EPW // C
EBLK = 1024
NBLK = 512

_INTERPRET = False


def _silu(x):
    return x * jax.nn.sigmoid(x)


def _sel_matrices():
    S1 = np.zeros((16, 32), np.float32)
    S2 = np.zeros((16, 32), np.float32)
    S3 = np.zeros((32, 16), np.float32)
    T1 = np.zeros((16, 32), np.float32)
    T2 = np.zeros((16, 32), np.float32)
    T3 = np.zeros((32, 16), np.float32)
    for i in range(3):
        for j in range(3):
            for k in range(3):
                m = i * 9 + j * 3 + k
                S1[j * 3 + i, m] = 1.0
                S2[j * 3 + k, m] = 1.0
                S3[m, i * 3 + k] = 1.0
                T1[i * 3 + j, m] = 1.0
                T2[j * 3 + k, m] = 1.0
                T3[m, i * 3 + k] = 1.0
    return S1, S2, S3, T1, T2, T3


_S1, _S2, _S3, _T1, _T2, _T3 = _sel_matrices()


def _k0_body(h_ref, wr_ref, wcb_ref, b1_ref, a_ref, b_ref):
    h = h_ref[...]
    a_ref[...] = jnp.dot(h, wr_ref[...], preferred_element_type=jnp.float32)
    b_ref[...] = jnp.dot(h, wcb_ref[...], preferred_element_type=jnp.float32) + b1_ref[...]


def _k0(hp, wr, wc, b1):
    grid = NP // NBLK
    return pl.pallas_call(
        _k0_body,
        grid=(grid,),
        in_specs=[
            pl.BlockSpec((NBLK, H), lambda i: (i, 0)),
            pl.BlockSpec((H, H), lambda i: (0, 0)),
            pl.BlockSpec((H, H), lambda i: (0, 0)),
            pl.BlockSpec((1, H), lambda i: (0, 0)),
        ],
        out_specs=[
            pl.BlockSpec((NBLK, H), lambda i: (i, 0)),
            pl.BlockSpec((NBLK, H), lambda i: (i, 0)),
        ],
        out_shape=[
            jax.ShapeDtypeStruct((NP, H), jnp.float32),
            jax.ShapeDtypeStruct((NP, H), jnp.float32),
        ],
        interpret=_INTERPRET,
    )(hp, wr, wc, b1)


def _k1a_body(row3, col3, a_hbm, b_hbm,
              ag_o, bg_o,
              idxr, idxc, abuf, bbuf,
              gsem0, gsem1, wsem):
    wid = lax.axis_index("s") * 2 + lax.axis_index("c")
    base = wid * EPW
    pltpu.sync_copy(row3.at[wid], idxr)
    pltpu.sync_copy(col3.at[wid], idxc)

    def gather(j, b, sem):
        return [
            pltpu.async_copy(a_hbm.at[idxr.at[j]], abuf.at[b], sem),
            pltpu.async_copy(b_hbm.at[idxc.at[j]], bbuf.at[b], sem),
        ]

    def writeback(j, b):
        off = base + j * C
        return [
            pltpu.async_copy(abuf.at[b], ag_o.at[pl.ds(off, C)], wsem),
            pltpu.async_copy(bbuf.at[b], bg_o.at[pl.ds(off, C)], wsem),
        ]

    def pair(k, carry):
        j0 = 2 * k
        j1 = 2 * k + 1
        g0 = gather(j0, 0, gsem0)
        g1 = gather(j1, 1, gsem1)
        for cp in g0:
            cp.wait()
        w0 = writeback(j0, 0)
        for cp in g1:
            cp.wait()
        w1 = writeback(j1, 1)
        for cp in w0 + w1:
            cp.wait()
        return carry

    lax.fori_loop(0, NCH // 2, pair, 0)


def _k1a(row3, col3, A, B):
    f32 = jnp.float32
    mesh = plsc.VectorSubcoreMesh(core_axis_name="c", subcore_axis_name="s")
    fn = functools.partial(
        pl.kernel,
        out_type=[
            jax.ShapeDtypeStruct((EP, H), f32),
            jax.ShapeDtypeStruct((EP, H), f32),
        ],
        mesh=mesh,
        scratch_types=[
            pltpu.VMEM((NCH, C), jnp.int32),
            pltpu.VMEM((NCH, C), jnp.int32),
            pltpu.VMEM((2, C, H), f32),
            pltpu.VMEM((2, C, H), f32),
            pltpu.SemaphoreType.DMA,
            pltpu.SemaphoreType.DMA,
            pltpu.SemaphoreType.DMA,
        ],
    )(_k1a_body)
    return fn(row3, col3, A, B)


def _k1b_body(row3, col3, vec_hbm,
              zi_o, zj_o,
              idxr, idxc, zibuf, zjbuf,
              gsem0, gsem1, wsem):
    wid = lax.axis_index("s") * 2 + lax.axis_index("c")
    base = wid * EPW
    pltpu.sync_copy(row3.at[wid], idxr)
    pltpu.sync_copy(col3.at[wid], idxc)

    def gather(j, b, sem):
        return [
            pltpu.async_copy(vec_hbm.at[idxr.at[j]], zibuf.at[b], sem),
            pltpu.async_copy(vec_hbm.at[idxc.at[j]], zjbuf.at[b], sem),
        ]

    def writeback(j, b):
        off = base + j * C
        return [
            pltpu.async_copy(zibuf.at[b], zi_o.at[pl.ds(off, C)], wsem),
            pltpu.async_copy(zjbuf.at[b], zj_o.at[pl.ds(off, C)], wsem),
        ]

    def pair(k, carry):
        j0 = 2 * k
        j1 = 2 * k + 1
        g0 = gather(j0, 0, gsem0)
        g1 = gather(j1, 1, gsem1)
        for cp in g0:
            cp.wait()
        w0 = writeback(j0, 0)
        for cp in g1:
            cp.wait()
        w1 = writeback(j1, 1)
        for cp in w0 + w1:
            cp.wait()
        return carry

    lax.fori_loop(0, NCH // 2, pair, 0)


def _k1b(row3, col3, vecp):
    f32 = jnp.float32
    mesh = plsc.VectorSubcoreMesh(core_axis_name="c", subcore_axis_name="s")
    fn = functools.partial(
        pl.kernel,
        out_type=[
            jax.ShapeDtypeStruct((EP, 16), f32),
            jax.ShapeDtypeStruct((EP, 16), f32),
        ],
        mesh=mesh,
        scratch_types=[
            pltpu.VMEM((NCH, C), jnp.int32),
            pltpu.VMEM((NCH, C), jnp.int32),
            pltpu.VMEM((2, C, 16), f32),
            pltpu.VMEM((2, C, 16), f32),
            pltpu.SemaphoreType.DMA,
            pltpu.SemaphoreType.DMA,
            pltpu.SemaphoreType.DMA,
        ],
        compiler_params=pltpu.CompilerParams(use_tc_tiling_on_sc=False),
    )(_k1b_body)
    return fn(row3, col3, vecp)


NPS = NP // 16


def _make_k3(width, tiled):
    f32 = jnp.float32

    def body(row3, val_hbm, zeros_hbm, acc_o, idx, vbuf0, vbuf1, acc_sh,
             lsem0, lsem1):
        cid = lax.axis_index("c")
        sid = lax.axis_index("s")
        wid = sid * 2 + cid
        base = wid * EPW
        pltpu.sync_copy(row3.at[wid], idx)
        pltpu.sync_copy(zeros_hbm, acc_sh.at[pl.ds(sid * NPS, NPS)])
        plsc.subcore_barrier()

        def pair(k, carry):
            j0 = 2 * k
            j1 = 2 * k + 1
            cp0 = pltpu.async_copy(val_hbm.at[pl.ds(base + j0 * C, C)], vbuf0, lsem0)
            cp1 = pltpu.async_copy(val_hbm.at[pl.ds(base + j1 * C, C)], vbuf1, lsem1)
            cp0.wait()
            pltpu.sync_copy(vbuf0, acc_sh.at[idx.at[j0]], add=True)
            cp1.wait()
            pltpu.sync_copy(vbuf1, acc_sh.at[idx.at[j1]], add=True)
            return carry

        lax.fori_loop(0, NCH // 2, pair, 0)
        plsc.subcore_barrier()
        r = sid * NPS
        pltpu.sync_copy(acc_sh.at[pl.ds(r, NPS)], acc_o.at[cid, pl.ds(r, NPS)])

    params = None if tiled else pltpu.CompilerParams(use_tc_tiling_on_sc=False)
    mesh = plsc.VectorSubcoreMesh(core_axis_name="c", subcore_axis_name="s")
    fn = functools.partial(
        pl.kernel,
        out_type=jax.ShapeDtypeStruct((2, NP, width), f32),
        mesh=mesh,
        scratch_types=[
            pltpu.VMEM((NCH, C), jnp.int32),
            pltpu.VMEM((C, width), f32),
            pltpu.VMEM((C, width), f32),
            pltpu.VMEM_SHARED((NP, width), f32),
            pltpu.SemaphoreType.DMA,
            pltpu.SemaphoreType.DMA,
        ],
        compiler_params=params,
    )(body)
    return fn


def _k2_body(ag_ref, bg_ref, zi_ref, zj_ref, fea_ref,
             wsf_ref, w2_ref, b2_ref, ow1_ref, ob1_ref, ow2_ref, ob2_ref,
             s1_ref, s2_ref, s3_ref, t1_ref, t2_ref, t3_ref,
             msg_ref, tail_ref):
    zi = zi_ref[...]
    zj = zj_ref[...]
    t = (jnp.dot(zj, s1_ref[...], preferred_element_type=jnp.float32)
         * jnp.dot(zi, s2_ref[...], preferred_element_type=jnp.float32))
    scal = jnp.dot(t, s3_ref[...], preferred_element_type=jnp.float32)
    ss = jnp.sum(scal * scal, axis=1, keepdims=True)
    nrm = jnp.sqrt(ss)
    scal = scal * (1.0 / jnp.maximum(nrm, 1e-12))

    u = jnp.concatenate([scal, fea_ref[...]], axis=1)
    x = _silu(ag_ref[...] + bg_ref[...]
              + jnp.dot(u, wsf_ref[...], preferred_element_type=jnp.float32))
    msg = _silu(jnp.dot(x.astype(jnp.bfloat16), w2_ref[...],
                        preferred_element_type=jnp.float32) + b2_ref[...])

    y = _silu(jnp.dot(msg.astype(jnp.bfloat16), ow1_ref[...],
                      preferred_element_type=jnp.float32) + ob1_ref[...])
    vs = jnp.dot(y.astype(jnp.bfloat16), ow2_ref[...],
                 preferred_element_type=jnp.float32) + ob2_ref[...]

    t2 = (jnp.dot(zj, t1_ref[...], preferred_element_type=jnp.float32)
          * jnp.dot(vs, t2_ref[...], preferred_element_type=jnp.float32))
    vec = jnp.dot(t2, t3_ref[...], preferred_element_type=jnp.float32)

    eid = pl.program_id(0) * EBLK + lax.broadcasted_iota(jnp.int32, (EBLK, 1), 0)
    maskf = jnp.where(eid < E, 1.0, 0.0)

    msg_ref[...] = msg * maskf
    tail = jnp.concatenate(
        [vec[:, :9], jnp.ones((EBLK, 1), jnp.float32),
         jnp.zeros((EBLK, 6), jnp.float32)], axis=1)
    tail_ref[...] = tail * maskf


def _k2(ag, bg, zi, zj, fea, wsf, w2, b2, ow1, ob1, ow2, ob2):
    grid = EP // EBLK
    full = lambda i: (0, 0)
    blk = lambda i: (i, 0)
    return pl.pallas_call(
        _k2_body,
        grid=(grid,),
        in_specs=[
            pl.BlockSpec((EBLK, H), blk),
            pl.BlockSpec((EBLK, H), blk),
            pl.BlockSpec((EBLK, 16), blk),
            pl.BlockSpec((EBLK, 16), blk),
            pl.BlockSpec((EBLK, 16), blk),
            pl.BlockSpec((32, H), full),
            pl.BlockSpec((H, H), full),
            pl.BlockSpec((1, H), full),
            pl.BlockSpec((H, H), full),
            pl.BlockSpec((1, H), full),
            pl.BlockSpec((H, 16), full),
            pl.BlockSpec((1, 16), full),
            pl.BlockSpec((16, 32), full),
            pl.BlockSpec((16, 32), full),
            pl.BlockSpec((32, 16), full),
            pl.BlockSpec((16, 32), full),
            pl.BlockSpec((16, 32), full),
            pl.BlockSpec((32, 16), full),
        ],
        out_specs=[
            pl.BlockSpec((EBLK, H), blk),
            pl.BlockSpec((EBLK, 16), blk),
        ],
        out_shape=[
            jax.ShapeDtypeStruct((EP, H), jnp.float32),
            jax.ShapeDtypeStruct((EP, 16), jnp.float32),
        ],
        interpret=_INTERPRET,
    )(ag, bg, zi, zj, fea, wsf, w2, b2, ow1, ob1, ow2, ob2,
      jnp.asarray(_S1), jnp.asarray(_S2), jnp.asarray(_S3),
      jnp.asarray(_T1), jnp.asarray(_T2), jnp.asarray(_T3))


def _k4_body(h_ref, accm_ref, accv_ref, vecp_ref,
             wn1a_ref, wn1b_ref, nb1_ref, wn2_ref, nb2_ref,
             hnew_ref, vout_ref):
    h = h_ref[...]
    tot = accm_ref[0] + accm_ref[1]
    vsum = accv_ref[0] + accv_ref[1]
    cnt = vsum[:, 9:10]
    recip = 1.0 / jnp.maximum(cnt, 1.0)
    vout_ref[...] = vsum * recip + vecp_ref[...]
    z = _silu(jnp.dot(h, wn1a_ref[...], preferred_element_type=jnp.float32)
              + jnp.dot(tot, wn1b_ref[...], preferred_element_type=jnp.float32)
              + nb1_ref[...])
    hnew_ref[...] = jnp.dot(z, wn2_ref[...], preferred_element_type=jnp.float32) + nb2_ref[...] + h


def _k4(hp, accm, accv, vecp, wn1a, wn1b, nb1, wn2, nb2):
    grid = NP // NBLK
    full = lambda i: (0, 0)
    return pl.pallas_call(
        _k4_body,
        grid=(grid,),
        in_specs=[
            pl.BlockSpec((NBLK, H), lambda i: (i, 0)),
            pl.BlockSpec((2, NBLK, H), lambda i: (0, i, 0)),
            pl.BlockSpec((2, NBLK, 16), lambda i: (0, i, 0)),
            pl.BlockSpec((NBLK, 16), lambda i: (i, 0)),
            pl.BlockSpec((H, H), full),
            pl.BlockSpec((H, H), full),
            pl.BlockSpec((1, H), full),
            pl.BlockSpec((H, H), full),
            pl.BlockSpec((1, H), full),
        ],
        out_specs=[
            pl.BlockSpec((NBLK, H), lambda i: (i, 0)),
            pl.BlockSpec((NBLK, 16), lambda i: (i, 0)),
        ],
        out_shape=[
            jax.ShapeDtypeStruct((NP, H), jnp.float32),
            jax.ShapeDtypeStruct((NP, 16), jnp.float32),
        ],
        interpret=_INTERPRET,
    )(hp, accm, accv, vecp, wn1a, wn1b, nb1, wn2, nb2)


def kernel(vectors, h, edge_index, edge_fea,
           in_w1, in_b1, in_w2, in_b2,
           out_w1, out_b1, out_w2, out_b2,
           node_w1, node_b1, node_w2, node_b2):
    f32 = jnp.float32
    row = edge_index[0].astype(jnp.int32)
    col = edge_index[1].astype(jnp.int32)
    rowp = jnp.pad(row, (0, EP - E))
    colp = jnp.pad(col, (0, EP - E))
    feap = jnp.pad(edge_fea.astype(f32), ((0, EP - E), (0, 16 - 4)))

    hp = jnp.pad(h.astype(f32), ((0, NP - N), (0, 0)))
    vecp = jnp.pad(vectors.astype(f32).reshape(N, V * V), ((0, NP - N), (0, 16 - V * V)))

    wr = in_w1[9:137]
    wc = in_w1[137:265]
    wsf = jnp.zeros((32, H), f32).at[0:9].set(in_w1[0:9]).at[16:20].set(in_w1[265:269])
    b1 = in_b1.reshape(1, H)

    A, B = _k0(hp, wr, wc, b1)

    row3 = rowp.reshape(NW, NCH, C)
    col3 = colp.reshape(NW, NCH, C)
    ag, bg = _k1a(row3, col3, A, B)
    zi, zj = _k1b(row3, col3, vecp)

    ow2p = jnp.zeros((H, 16), f32).at[:, :9].set(out_w2)
    ob2p = jnp.zeros((1, 16), f32).at[0, :9].set(out_b2)

    msg, tail = _k2(ag, bg, zi, zj, feap,
                    wsf, in_w2.astype(jnp.bfloat16), in_b2.reshape(1, H),
                    out_w1.astype(jnp.bfloat16), out_b1.reshape(1, H),
                    ow2p.astype(jnp.bfloat16), ob2p)

    accm = _make_k3(H, True)(row3, msg, jnp.zeros((NPS, H), f32))
    accv = _make_k3(16, False)(row3, tail, jnp.zeros((NPS, 16), f32))

    wn1a = node_w1[:H]
    wn1b = node_w1[H:]

    hnew, vout = _k4(hp, accm, accv, vecp,
                     wn1a, wn1b, node_b1.reshape(1, H),
                     node_w2, node_b2.reshape(1, H))

    return (vout[:N, :9].reshape(N, 3, V), hnew[:N])


# speedup vs baseline: 3.8480x; 1.0127x over previous
import functools
import numpy as np
import jax
import jax.numpy as jnp
from jax import lax
from jax.experimental import pallas as pl
from jax.experimental.pallas import tpu as pltpu
from jax.experimental.pallas import tpu_sc as plsc

N = 10000
E = 320000
V = 3
H = 128

NP = 10240
NW = 32
C = 128
EP = 327680
EPW = EP // NW
NCH = EPW // C
EBLK = 2048
NBLK = 512

_INTERPRET = False


def _silu(x):
    return x * jax.nn.sigmoid(x)


def _sel_matrices():
    S1 = np.zeros((16, 32), np.float32)
    S2 = np.zeros((16, 32), np.float32)
    S3 = np.zeros((32, 16), np.float32)
    T1 = np.zeros((16, 32), np.float32)
    T2 = np.zeros((16, 32), np.float32)
    T3 = np.zeros((32, 16), np.float32)
    for i in range(3):
        for j in range(3):
            for k in range(3):
                m = i * 9 + j * 3 + k
                S1[j * 3 + i, m] = 1.0
                S2[j * 3 + k, m] = 1.0
                S3[m, i * 3 + k] = 1.0
                T1[i * 3 + j, m] = 1.0
                T2[j * 3 + k, m] = 1.0
                T3[m, i * 3 + k] = 1.0
    return S1, S2, S3, T1, T2, T3


_S1, _S2, _S3, _T1, _T2, _T3 = _sel_matrices()


def _k0_body(h_ref, wr_ref, wcb_ref, b1_ref, a_ref, b_ref):
    h = h_ref[...]
    a_ref[...] = jnp.dot(h, wr_ref[...],
                         preferred_element_type=jnp.float32).astype(jnp.bfloat16)
    b_ref[...] = (jnp.dot(h, wcb_ref[...], preferred_element_type=jnp.float32)
                  + b1_ref[...]).astype(jnp.bfloat16)


def _k0(hp, wr, wc, b1):
    grid = NP // NBLK
    return pl.pallas_call(
        _k0_body,
        grid=(grid,),
        in_specs=[
            pl.BlockSpec((NBLK, H), lambda i: (i, 0)),
            pl.BlockSpec((H, H), lambda i: (0, 0)),
            pl.BlockSpec((H, H), lambda i: (0, 0)),
            pl.BlockSpec((1, H), lambda i: (0, 0)),
        ],
        out_specs=[
            pl.BlockSpec((NBLK, H), lambda i: (i, 0)),
            pl.BlockSpec((NBLK, H), lambda i: (i, 0)),
        ],
        out_shape=[
            jax.ShapeDtypeStruct((NP, H), jnp.bfloat16),
            jax.ShapeDtypeStruct((NP, H), jnp.bfloat16),
        ],
        interpret=_INTERPRET,
    )(hp, wr, wc, b1)


def _k1_body(row3, col3, a_hbm, b_hbm, vec_hbm,
             ag_o, bg_o, zi_o, zj_o,
             idxr, idxc, abuf, bbuf, zibuf, zjbuf,
             gsem0, gsem1, wsem):
    wid = lax.axis_index("s") * 2 + lax.axis_index("c")
    base = wid * EPW
    pltpu.sync_copy(row3.at[wid], idxr)
    pltpu.sync_copy(col3.at[wid], idxc)

    def gather(j, b, sem):
        return [
            pltpu.async_copy(a_hbm.at[idxr.at[j]], abuf.at[b], sem),
            pltpu.async_copy(b_hbm.at[idxc.at[j]], bbuf.at[b], sem),
            pltpu.async_copy(vec_hbm.at[idxr.at[j]], zibuf.at[b], sem),
            pltpu.async_copy(vec_hbm.at[idxc.at[j]], zjbuf.at[b], sem),
        ]

    def writeback(j, b):
        off = base + j * C
        return [
            pltpu.async_copy(abuf.at[b], ag_o.at[pl.ds(off, C)], wsem),
            pltpu.async_copy(bbuf.at[b], bg_o.at[pl.ds(off, C)], wsem),
            pltpu.async_copy(zibuf.at[b], zi_o.at[pl.ds(off, C)], wsem),
            pltpu.async_copy(zjbuf.at[b], zj_o.at[pl.ds(off, C)], wsem),
        ]

    def pair(k, carry):
        j0 = 2 * k
        j1 = 2 * k + 1
        g0 = gather(j0, 0, gsem0)
        g1 = gather(j1, 1, gsem1)
        for cp in g0:
            cp.wait()
        w0 = writeback(j0, 0)
        for cp in g1:
            cp.wait()
        w1 = writeback(j1, 1)
        for cp in w0 + w1:
            cp.wait()
        return carry

    lax.fori_loop(0, NCH // 2, pair, 0)


def _k1(row3, col3, A, B, vecp):
    f32 = jnp.float32
    bf16 = jnp.bfloat16
    mesh = plsc.VectorSubcoreMesh(core_axis_name="c", subcore_axis_name="s")
    fn = functools.partial(
        pl.kernel,
        out_type=[
            jax.ShapeDtypeStruct((EP, H), bf16),
            jax.ShapeDtypeStruct((EP, H), bf16),
            jax.ShapeDtypeStruct((EP, 16), f32),
            jax.ShapeDtypeStruct((EP, 16), f32),
        ],
        mesh=mesh,
        scratch_types=[
            pltpu.VMEM((NCH, C), jnp.int32),
            pltpu.VMEM((NCH, C), jnp.int32),
            pltpu.VMEM((2, C, H), bf16),
            pltpu.VMEM((2, C, H), bf16),
            pltpu.VMEM((2, C, 16), f32),
            pltpu.VMEM((2, C, 16), f32),
            pltpu.SemaphoreType.DMA,
            pltpu.SemaphoreType.DMA,
            pltpu.SemaphoreType.DMA,
        ],
        compiler_params=pltpu.CompilerParams(use_tc_tiling_on_sc=False),
    )(_k1_body)
    return fn(row3, col3, A, B, vecp)


NPS = NP // 16


def _make_k3(width, tiled):
    f32 = jnp.float32

    def body(row3, val_hbm, zeros_hbm, acc_o, idx, vbuf0, vbuf1, acc_sh,
             lsem0, lsem1):
        cid = lax.axis_index("c")
        sid = lax.axis_index("s")
        wid = sid * 2 + cid
        base = wid * EPW
        pltpu.sync_copy(row3.at[wid], idx)
        pltpu.sync_copy(zeros_hbm, acc_sh.at[pl.ds(sid * NPS, NPS)])
        plsc.subcore_barrier()

        def pair(k, carry):
            j0 = 2 * k
            j1 = 2 * k + 1
            cp0 = pltpu.async_copy(val_hbm.at[pl.ds(base + j0 * C, C)], vbuf0, lsem0)
            cp1 = pltpu.async_copy(val_hbm.at[pl.ds(base + j1 * C, C)], vbuf1, lsem1)
            cp0.wait()
            pltpu.sync_copy(vbuf0, acc_sh.at[idx.at[j0]], add=True)
            cp1.wait()
            pltpu.sync_copy(vbuf1, acc_sh.at[idx.at[j1]], add=True)
            return carry

        lax.fori_loop(0, NCH // 2, pair, 0)
        plsc.subcore_barrier()
        r = sid * NPS
        pltpu.sync_copy(acc_sh.at[pl.ds(r, NPS)], acc_o.at[cid, pl.ds(r, NPS)])

    params = None if tiled else pltpu.CompilerParams(use_tc_tiling_on_sc=False)
    mesh = plsc.VectorSubcoreMesh(core_axis_name="c", subcore_axis_name="s")
    fn = functools.partial(
        pl.kernel,
        out_type=jax.ShapeDtypeStruct((2, NP, width), f32),
        mesh=mesh,
        scratch_types=[
            pltpu.VMEM((NCH, C), jnp.int32),
            pltpu.VMEM((C, width), f32),
            pltpu.VMEM((C, width), f32),
            pltpu.VMEM_SHARED((NP, width), f32),
            pltpu.SemaphoreType.DMA,
            pltpu.SemaphoreType.DMA,
        ],
        compiler_params=params,
    )(body)
    return fn


def _k2_body(ag_ref, bg_ref, zi_ref, zj_ref, fea_ref,
             wsf_ref, w2_ref, b2_ref, ow1_ref, ob1_ref, ow2_ref, ob2_ref,
             s1_ref, s2_ref, s3_ref, t1_ref, t2_ref, t3_ref, cnt_ref,
             msg_ref, tail_ref):
    zi = zi_ref[...]
    zj = zj_ref[...]
    t = (jnp.dot(zj, s1_ref[...], preferred_element_type=jnp.float32)
         * jnp.dot(zi, s2_ref[...], preferred_element_type=jnp.float32))
    scal = jnp.dot(t, s3_ref[...], preferred_element_type=jnp.float32)
    ss = jnp.sum(scal * scal, axis=1, keepdims=True)
    nrm = jnp.sqrt(ss)
    scal = scal * (1.0 / jnp.maximum(nrm, 1e-12))

    u = jnp.concatenate([scal, fea_ref[...]], axis=1)
    x = _silu(ag_ref[...].astype(jnp.float32) + bg_ref[...].astype(jnp.float32)
              + jnp.dot(u, wsf_ref[...], preferred_element_type=jnp.float32))
    msg = _silu(jnp.dot(x.astype(jnp.bfloat16), w2_ref[...],
                        preferred_element_type=jnp.float32) + b2_ref[...])

    y = _silu(jnp.dot(msg.astype(jnp.bfloat16), ow1_ref[...],
                      preferred_element_type=jnp.float32) + ob1_ref[...])
    vs = jnp.dot(y.astype(jnp.bfloat16), ow2_ref[...],
                 preferred_element_type=jnp.float32) + ob2_ref[...]

    t2 = (jnp.dot(zj, t1_ref[...], preferred_element_type=jnp.float32)
          * jnp.dot(vs, t2_ref[...], preferred_element_type=jnp.float32))
    vec = jnp.dot(t2, t3_ref[...], preferred_element_type=jnp.float32)

    msg_ref[...] = msg
    tail_ref[...] = vec + cnt_ref[...]


def _k2(ag, bg, zi, zj, fea, wsf, w2, b2, ow1, ob1, ow2, ob2):
    grid = EP // EBLK
    full = lambda i: (0, 0)
    blk = lambda i: (i, 0)
    cnt = np.zeros((1, 16), np.float32)
    cnt[0, 9] = 1.0
    return pl.pallas_call(
        _k2_body,
        grid=(grid,),
        in_specs=[
            pl.BlockSpec((EBLK, H), blk),
            pl.BlockSpec((EBLK, H), blk),
            pl.BlockSpec((EBLK, 16), blk),
            pl.BlockSpec((EBLK, 16), blk),
            pl.BlockSpec((EBLK, 16), blk),
            pl.BlockSpec((32, H), full),
            pl.BlockSpec((H, H), full),
            pl.BlockSpec((1, H), full),
            pl.BlockSpec((H, H), full),
            pl.BlockSpec((1, H), full),
            pl.BlockSpec((H, 16), full),
            pl.BlockSpec((1, 16), full),
            pl.BlockSpec((16, 32), full),
            pl.BlockSpec((16, 32), full),
            pl.BlockSpec((32, 16), full),
            pl.BlockSpec((16, 32), full),
            pl.BlockSpec((16, 32), full),
            pl.BlockSpec((32, 16), full),
            pl.BlockSpec((1, 16), full),
        ],
        out_specs=[
            pl.BlockSpec((EBLK, H), blk),
            pl.BlockSpec((EBLK, 16), blk),
        ],
        out_shape=[
            jax.ShapeDtypeStruct((EP, H), jnp.float32),
            jax.ShapeDtypeStruct((EP, 16), jnp.float32),
        ],
        interpret=_INTERPRET,
    )(ag, bg, zi, zj, fea, wsf, w2, b2, ow1, ob1, ow2, ob2,
      jnp.asarray(_S1), jnp.asarray(_S2), jnp.asarray(_S3),
      jnp.asarray(_T1), jnp.asarray(_T2), jnp.asarray(_T3),
      jnp.asarray(cnt))


def _k4_body(h_ref, accm_ref, accv_ref, vecp_ref,
             wn1a_ref, wn1b_ref, nb1_ref, wn2_ref, nb2_ref,
             hnew_ref, vout_ref):
    h = h_ref[...]
    tot = accm_ref[0] + accm_ref[1]
    vsum = accv_ref[0] + accv_ref[1]
    cnt = vsum[:, 9:10]
    recip = 1.0 / jnp.maximum(cnt, 1.0)
    vout_ref[...] = vsum * recip + vecp_ref[...]
    z = _silu(jnp.dot(h, wn1a_ref[...], preferred_element_type=jnp.float32)
              + jnp.dot(tot, wn1b_ref[...], preferred_element_type=jnp.float32)
              + nb1_ref[...])
    hnew_ref[...] = jnp.dot(z, wn2_ref[...], preferred_element_type=jnp.float32) + nb2_ref[...] + h


def _k4(hp, accm, accv, vecp, wn1a, wn1b, nb1, wn2, nb2):
    grid = NP // NBLK
    full = lambda i: (0, 0)
    return pl.pallas_call(
        _k4_body,
        grid=(grid,),
        in_specs=[
            pl.BlockSpec((NBLK, H), lambda i: (i, 0)),
            pl.BlockSpec((2, NBLK, H), lambda i: (0, i, 0)),
            pl.BlockSpec((2, NBLK, 16), lambda i: (0, i, 0)),
            pl.BlockSpec((NBLK, 16), lambda i: (i, 0)),
            pl.BlockSpec((H, H), full),
            pl.BlockSpec((H, H), full),
            pl.BlockSpec((1, H), full),
            pl.BlockSpec((H, H), full),
            pl.BlockSpec((1, H), full),
        ],
        out_specs=[
            pl.BlockSpec((NBLK, H), lambda i: (i, 0)),
            pl.BlockSpec((NBLK, 16), lambda i: (i, 0)),
        ],
        out_shape=[
            jax.ShapeDtypeStruct((NP, H), jnp.float32),
            jax.ShapeDtypeStruct((NP, 16), jnp.float32),
        ],
        interpret=_INTERPRET,
    )(hp, accm, accv, vecp, wn1a, wn1b, nb1, wn2, nb2)


def kernel(vectors, h, edge_index, edge_fea,
           in_w1, in_b1, in_w2, in_b2,
           out_w1, out_b1, out_w2, out_b2,
           node_w1, node_b1, node_w2, node_b2):
    f32 = jnp.float32
    row = edge_index[0].astype(jnp.int32)
    col = edge_index[1].astype(jnp.int32)
    rowp = jnp.pad(row, (0, EP - E), constant_values=NP - 1)
    colp = jnp.pad(col, (0, EP - E))
    feap = jnp.pad(edge_fea.astype(f32), ((0, EP - E), (0, 16 - 4)))

    hp = jnp.pad(h.astype(f32), ((0, NP - N), (0, 0)))
    vecp = jnp.pad(vectors.astype(f32).reshape(N, V * V), ((0, NP - N), (0, 16 - V * V)))

    wr = in_w1[9:137]
    wc = in_w1[137:265]
    wsf = jnp.zeros((32, H), f32).at[0:9].set(in_w1[0:9]).at[16:20].set(in_w1[265:269])
    b1 = in_b1.reshape(1, H)

    A, B = _k0(hp, wr, wc, b1)

    row3 = rowp.reshape(NW, NCH, C)
    col3 = colp.reshape(NW, NCH, C)
    ag, bg, zi, zj = _k1(row3, col3, A, B, vecp)

    ow2p = jnp.zeros((H, 16), f32).at[:, :9].set(out_w2)
    ob2p = jnp.zeros((1, 16), f32).at[0, :9].set(out_b2)

    msg, tail = _k2(ag, bg, zi, zj, feap,
                    wsf, in_w2.astype(jnp.bfloat16), in_b2.reshape(1, H),
                    out_w1.astype(jnp.bfloat16), out_b1.reshape(1, H),
                    ow2p.astype(jnp.bfloat16), ob2p)

    accm = _make_k3(H, True)(row3, msg, jnp.zeros((NPS, H), f32))
    accv = _make_k3(16, False)(row3, tail, jnp.zeros((NPS, 16), f32))

    wn1a = node_w1[:H]
    wn1b = node_w1[H:]

    hnew, vout = _k4(hp, accm, accv, vecp,
                     wn1a, wn1b, node_b1.reshape(1, H),
                     node_w2, node_b2.reshape(1, H))

    return (vout[:N, :9].reshape(N, 3, V), hnew[:N])


# speedup vs baseline: 3.9096x; 1.0160x over previous
import functools
import numpy as np
import jax
import jax.numpy as jnp
from jax import lax
from jax.experimental import pallas as pl
from jax.experimental.pallas import tpu as pltpu
from jax.experimental.pallas import tpu_sc as plsc

N = 10000
E = 320000
V = 3
H = 128

NP = 10240
NW = 32
C = 128
EP = 327680
EP2 = EP // 2
EPW = EP2 // NW
NCH = EPW // C
EBLK = 2048
NBLK = 512

_INTERPRET = False


def _silu(x):
    return x * jax.nn.sigmoid(x)


def _sel_matrices():
    S1 = np.zeros((16, 32), np.float32)
    S2 = np.zeros((16, 32), np.float32)
    S3 = np.zeros((32, 16), np.float32)
    T1 = np.zeros((16, 32), np.float32)
    T2 = np.zeros((16, 32), np.float32)
    T3 = np.zeros((32, 16), np.float32)
    for i in range(3):
        for j in range(3):
            for k in range(3):
                m = i * 9 + j * 3 + k
                S1[j * 3 + i, m] = 1.0
                S2[j * 3 + k, m] = 1.0
                S3[m, i * 3 + k] = 1.0
                T1[i * 3 + j, m] = 1.0
                T2[j * 3 + k, m] = 1.0
                T3[m, i * 3 + k] = 1.0
    return S1, S2, S3, T1, T2, T3


_S1, _S2, _S3, _T1, _T2, _T3 = _sel_matrices()


def _k0_body(h_ref, wr_ref, wcb_ref, b1_ref, a_ref, b_ref):
    h = h_ref[...]
    a_ref[...] = jnp.dot(h, wr_ref[...],
                         preferred_element_type=jnp.float32).astype(jnp.bfloat16)
    b_ref[...] = (jnp.dot(h, wcb_ref[...], preferred_element_type=jnp.float32)
                  + b1_ref[...]).astype(jnp.bfloat16)


def _k0(hp, wr, wc, b1):
    grid = NP // NBLK
    return pl.pallas_call(
        _k0_body,
        grid=(grid,),
        in_specs=[
            pl.BlockSpec((NBLK, H), lambda i: (i, 0)),
            pl.BlockSpec((H, H), lambda i: (0, 0)),
            pl.BlockSpec((H, H), lambda i: (0, 0)),
            pl.BlockSpec((1, H), lambda i: (0, 0)),
        ],
        out_specs=[
            pl.BlockSpec((NBLK, H), lambda i: (i, 0)),
            pl.BlockSpec((NBLK, H), lambda i: (i, 0)),
        ],
        out_shape=[
            jax.ShapeDtypeStruct((NP, H), jnp.bfloat16),
            jax.ShapeDtypeStruct((NP, H), jnp.bfloat16),
        ],
        interpret=_INTERPRET,
    )(hp, wr, wc, b1)


def _k1_body(row3, col3, a_hbm, b_hbm, vec_hbm,
             ag_o, bg_o, zi_o, zj_o,
             idxr, idxc, abuf, bbuf, zibuf, zjbuf,
             gsem0, gsem1, wsem):
    wid = lax.axis_index("s") * 2 + lax.axis_index("c")
    base = wid * EPW
    pltpu.sync_copy(row3.at[wid], idxr)
    pltpu.sync_copy(col3.at[wid], idxc)

    def gather(j, b, sem):
        return [
            pltpu.async_copy(a_hbm.at[idxr.at[j]], abuf.at[b], sem),
            pltpu.async_copy(b_hbm.at[idxc.at[j]], bbuf.at[b], sem),
            pltpu.async_copy(vec_hbm.at[idxr.at[j]], zibuf.at[b], sem),
            pltpu.async_copy(vec_hbm.at[idxc.at[j]], zjbuf.at[b], sem),
        ]

    def writeback(j, b):
        off = base + j * C
        return [
            pltpu.async_copy(abuf.at[b], ag_o.at[pl.ds(off, C)], wsem),
            pltpu.async_copy(bbuf.at[b], bg_o.at[pl.ds(off, C)], wsem),
            pltpu.async_copy(zibuf.at[b], zi_o.at[pl.ds(off, C)], wsem),
            pltpu.async_copy(zjbuf.at[b], zj_o.at[pl.ds(off, C)], wsem),
        ]

    def pair(k, carry):
        j0 = 2 * k
        j1 = 2 * k + 1
        g0 = gather(j0, 0, gsem0)
        g1 = gather(j1, 1, gsem1)
        for cp in g0:
            cp.wait()
        w0 = writeback(j0, 0)
        for cp in g1:
            cp.wait()
        w1 = writeback(j1, 1)
        for cp in w0 + w1:
            cp.wait()
        return carry

    lax.fori_loop(0, NCH // 2, pair, 0)


def _k1(row3, col3, A, B, vecp):
    f32 = jnp.float32
    bf16 = jnp.bfloat16
    mesh = plsc.VectorSubcoreMesh(core_axis_name="c", subcore_axis_name="s")
    fn = functools.partial(
        pl.kernel,
        out_type=[
            jax.ShapeDtypeStruct((EP2, H), bf16),
            jax.ShapeDtypeStruct((EP2, H), bf16),
            jax.ShapeDtypeStruct((EP2, 16), f32),
            jax.ShapeDtypeStruct((EP2, 16), f32),
        ],
        mesh=mesh,
        scratch_types=[
            pltpu.VMEM((NCH, C), jnp.int32),
            pltpu.VMEM((NCH, C), jnp.int32),
            pltpu.VMEM((2, C, H), bf16),
            pltpu.VMEM((2, C, H), bf16),
            pltpu.VMEM((2, C, 16), f32),
            pltpu.VMEM((2, C, 16), f32),
            pltpu.SemaphoreType.DMA,
            pltpu.SemaphoreType.DMA,
            pltpu.SemaphoreType.DMA,
        ],
        compiler_params=pltpu.CompilerParams(use_tc_tiling_on_sc=False),
    )(_k1_body)
    return fn(row3, col3, A, B, vecp)


NPS = NP // 16


def _make_k3(width, tiled):
    f32 = jnp.float32

    def body(row3, val_hbm, zeros_hbm, acc_o, idx, vbuf0, vbuf1, acc_sh,
             lsem0, lsem1):
        cid = lax.axis_index("c")
        sid = lax.axis_index("s")
        wid = sid * 2 + cid
        base = wid * EPW
        pltpu.sync_copy(row3.at[wid], idx)
        pltpu.sync_copy(zeros_hbm, acc_sh.at[pl.ds(sid * NPS, NPS)])
        plsc.subcore_barrier()

        def pair(k, carry):
            j0 = 2 * k
            j1 = 2 * k + 1
            cp0 = pltpu.async_copy(val_hbm.at[pl.ds(base + j0 * C, C)], vbuf0, lsem0)
            cp1 = pltpu.async_copy(val_hbm.at[pl.ds(base + j1 * C, C)], vbuf1, lsem1)
            cp0.wait()
            pltpu.sync_copy(vbuf0, acc_sh.at[idx.at[j0]], add=True)
            cp1.wait()
            pltpu.sync_copy(vbuf1, acc_sh.at[idx.at[j1]], add=True)
            return carry

        lax.fori_loop(0, NCH // 2, pair, 0)
        plsc.subcore_barrier()
        r = sid * NPS
        pltpu.sync_copy(acc_sh.at[pl.ds(r, NPS)], acc_o.at[cid, pl.ds(r, NPS)])

    params = None if tiled else pltpu.CompilerParams(use_tc_tiling_on_sc=False)
    mesh = plsc.VectorSubcoreMesh(core_axis_name="c", subcore_axis_name="s")
    fn = functools.partial(
        pl.kernel,
        out_type=jax.ShapeDtypeStruct((2, NP, width), f32),
        name=f"k3_{width}",
        mesh=mesh,
        scratch_types=[
            pltpu.VMEM((NCH, C), jnp.int32),
            pltpu.VMEM((C, width), f32),
            pltpu.VMEM((C, width), f32),
            pltpu.VMEM_SHARED((NP, width), f32),
            pltpu.SemaphoreType.DMA,
            pltpu.SemaphoreType.DMA,
        ],
        compiler_params=params,
    )(body)
    return fn


def _k2_body(ag_ref, bg_ref, zi_ref, zj_ref, fea_ref,
             wsf_ref, w2_ref, b2_ref, ow1_ref, ob1_ref, ow2_ref, ob2_ref,
             s1_ref, s2_ref, s3_ref, t1_ref, t2_ref, t3_ref, cnt_ref,
             msg_ref, tail_ref):
    zi = zi_ref[...]
    zj = zj_ref[...]
    t = (jnp.dot(zj, s1_ref[...], preferred_element_type=jnp.float32)
         * jnp.dot(zi, s2_ref[...], preferred_element_type=jnp.float32))
    scal = jnp.dot(t, s3_ref[...], preferred_element_type=jnp.float32)
    ss = jnp.sum(scal * scal, axis=1, keepdims=True)
    nrm = jnp.sqrt(ss)
    scal = scal * (1.0 / jnp.maximum(nrm, 1e-12))

    u = jnp.concatenate([scal, fea_ref[...]], axis=1)
    x = _silu(ag_ref[...].astype(jnp.float32) + bg_ref[...].astype(jnp.float32)
              + jnp.dot(u, wsf_ref[...], preferred_element_type=jnp.float32))
    msg = _silu(jnp.dot(x.astype(jnp.bfloat16), w2_ref[...],
                        preferred_element_type=jnp.float32) + b2_ref[...])

    y = _silu(jnp.dot(msg.astype(jnp.bfloat16), ow1_ref[...],
                      preferred_element_type=jnp.float32) + ob1_ref[...])
    vs = jnp.dot(y.astype(jnp.bfloat16), ow2_ref[...],
                 preferred_element_type=jnp.float32) + ob2_ref[...]

    t2 = (jnp.dot(zj, t1_ref[...], preferred_element_type=jnp.float32)
          * jnp.dot(vs, t2_ref[...], preferred_element_type=jnp.float32))
    vec = jnp.dot(t2, t3_ref[...], preferred_element_type=jnp.float32)

    msg_ref[...] = msg
    tail_ref[...] = vec + cnt_ref[...]


def _k2(half, ag, bg, zi, zj, fea, wsf, w2, b2, ow1, ob1, ow2, ob2):
    grid = EP2 // EBLK
    hoff = half * grid
    full = lambda i: (0, 0)
    blk = lambda i: (i, 0)
    fblk = lambda i: (i + hoff, 0)
    cnt = np.zeros((1, 16), np.float32)
    cnt[0, 9] = 1.0
    return pl.pallas_call(
        _k2_body,
        grid=(grid,),
        in_specs=[
            pl.BlockSpec((EBLK, H), blk),
            pl.BlockSpec((EBLK, H), blk),
            pl.BlockSpec((EBLK, 16), blk),
            pl.BlockSpec((EBLK, 16), blk),
            pl.BlockSpec((EBLK, 16), fblk),
            pl.BlockSpec((32, H), full),
            pl.BlockSpec((H, H), full),
            pl.BlockSpec((1, H), full),
            pl.BlockSpec((H, H), full),
            pl.BlockSpec((1, H), full),
            pl.BlockSpec((H, 16), full),
            pl.BlockSpec((1, 16), full),
            pl.BlockSpec((16, 32), full),
            pl.BlockSpec((16, 32), full),
            pl.BlockSpec((32, 16), full),
            pl.BlockSpec((16, 32), full),
            pl.BlockSpec((16, 32), full),
            pl.BlockSpec((32, 16), full),
            pl.BlockSpec((1, 16), full),
        ],
        out_specs=[
            pl.BlockSpec((EBLK, H), blk),
            pl.BlockSpec((EBLK, 16), blk),
        ],
        out_shape=[
            jax.ShapeDtypeStruct((EP2, H), jnp.float32),
            jax.ShapeDtypeStruct((EP2, 16), jnp.float32),
        ],
        interpret=_INTERPRET,
    )(ag, bg, zi, zj, fea, wsf, w2, b2, ow1, ob1, ow2, ob2,
      jnp.asarray(_S1), jnp.asarray(_S2), jnp.asarray(_S3),
      jnp.asarray(_T1), jnp.asarray(_T2), jnp.asarray(_T3),
      jnp.asarray(cnt))


def _k4_body(h_ref, accm0_ref, accm1_ref, accv0_ref, accv1_ref, vecp_ref,
             wn1a_ref, wn1b_ref, nb1_ref, wn2_ref, nb2_ref,
             hnew_ref, vout_ref):
    h = h_ref[...]
    tot = (accm0_ref[0] + accm0_ref[1]
           + accm1_ref[0] + accm1_ref[1])
    vsum = (accv0_ref[0] + accv0_ref[1]
            + accv1_ref[0] + accv1_ref[1])
    cnt = vsum[:, 9:10]
    recip = 1.0 / jnp.maximum(cnt, 1.0)
    vout_ref[...] = vsum * recip + vecp_ref[...]
    z = _silu(jnp.dot(h, wn1a_ref[...], preferred_element_type=jnp.float32)
              + jnp.dot(tot, wn1b_ref[...], preferred_element_type=jnp.float32)
              + nb1_ref[...])
    hnew_ref[...] = jnp.dot(z, wn2_ref[...], preferred_element_type=jnp.float32) + nb2_ref[...] + h


def _k4(hp, accm0, accm1, accv0, accv1, vecp, wn1a, wn1b, nb1, wn2, nb2):
    grid = NP // NBLK
    full = lambda i: (0, 0)
    return pl.pallas_call(
        _k4_body,
        grid=(grid,),
        in_specs=[
            pl.BlockSpec((NBLK, H), lambda i: (i, 0)),
            pl.BlockSpec((2, NBLK, H), lambda i: (0, i, 0)),
            pl.BlockSpec((2, NBLK, H), lambda i: (0, i, 0)),
            pl.BlockSpec((2, NBLK, 16), lambda i: (0, i, 0)),
            pl.BlockSpec((2, NBLK, 16), lambda i: (0, i, 0)),
            pl.BlockSpec((NBLK, 16), lambda i: (i, 0)),
            pl.BlockSpec((H, H), full),
            pl.BlockSpec((H, H), full),
            pl.BlockSpec((1, H), full),
            pl.BlockSpec((H, H), full),
            pl.BlockSpec((1, H), full),
        ],
        out_specs=[
            pl.BlockSpec((NBLK, H), lambda i: (i, 0)),
            pl.BlockSpec((NBLK, 16), lambda i: (i, 0)),
        ],
        out_shape=[
            jax.ShapeDtypeStruct((NP, H), jnp.float32),
            jax.ShapeDtypeStruct((NP, 16), jnp.float32),
        ],
        interpret=_INTERPRET,
    )(hp, accm0, accm1, accv0, accv1, vecp, wn1a, wn1b, nb1, wn2, nb2)


def kernel(vectors, h, edge_index, edge_fea,
           in_w1, in_b1, in_w2, in_b2,
           out_w1, out_b1, out_w2, out_b2,
           node_w1, node_b1, node_w2, node_b2):
    f32 = jnp.float32
    row = edge_index[0].astype(jnp.int32)
    col = edge_index[1].astype(jnp.int32)
    rowp = jnp.pad(row, (0, EP - E), constant_values=NP - 1)
    colp = jnp.pad(col, (0, EP - E))
    feap = jnp.pad(edge_fea.astype(f32), ((0, EP - E), (0, 16 - 4)))

    hp = jnp.pad(h.astype(f32), ((0, NP - N), (0, 0)))
    vecp = jnp.pad(vectors.astype(f32).reshape(N, V * V), ((0, NP - N), (0, 16 - V * V)))

    wr = in_w1[9:137]
    wc = in_w1[137:265]
    wsf = jnp.zeros((32, H), f32).at[0:9].set(in_w1[0:9]).at[16:20].set(in_w1[265:269])
    b1 = in_b1.reshape(1, H)

    A, B = _k0(hp, wr, wc, b1)

    row4 = rowp.reshape(2, NW, NCH, C)
    col4 = colp.reshape(2, NW, NCH, C)

    ow2p = jnp.zeros((H, 16), f32).at[:, :9].set(out_w2)
    ob2p = jnp.zeros((1, 16), f32).at[0, :9].set(out_b2)
    w2b = in_w2.astype(jnp.bfloat16)
    ow1b = out_w1.astype(jnp.bfloat16)
    ow2b = ow2p.astype(jnp.bfloat16)
    zm = jnp.zeros((NPS, H), f32)
    zv = jnp.zeros((NPS, 16), f32)
    k3m = _make_k3(H, True)
    k3v = _make_k3(16, False)

    accms = []
    accvs = []
    for hlf in range(2):
        ag, bg, zi, zj = _k1(row4[hlf], col4[hlf], A, B, vecp)
        msg, tail = _k2(hlf, ag, bg, zi, zj, feap,
                        wsf, w2b, in_b2.reshape(1, H),
                        ow1b, out_b1.reshape(1, H), ow2b, ob2p)
        accms.append(k3m(row4[hlf], msg, zm))
        accvs.append(k3v(row4[hlf], tail, zv))

    wn1a = node_w1[:H]
    wn1b = node_w1[H:]

    hnew, vout = _k4(hp, accms[0], accms[1], accvs[0], accvs[1], vecp,
                     wn1a, wn1b, node_b1.reshape(1, H),
                     node_w2, node_b2.reshape(1, H))

    return (vout[:N, :9].reshape(N, 3, V), hnew[:N])


# speedup vs baseline: 4.6252x; 1.1830x over previous
import functools
import numpy as np
import jax
import jax.numpy as jnp
from jax import lax
from jax.experimental import pallas as pl
from jax.experimental.pallas import tpu as pltpu
from jax.experimental.pallas import tpu_sc as plsc

N = 10000
E = 320000
V = 3
H = 128

NP = 10240
NW = 32
C = 128
EP = 327680
EP2 = EP // 2
EPW = EP2 // NW
NCH = EPW // C
EBLK = 2048
NBLK = 512

_INTERPRET = False


def _silu(x):
    return x * jax.nn.sigmoid(x)


def _sel_matrices():
    S1 = np.zeros((16, 32), np.float32)
    S2 = np.zeros((16, 32), np.float32)
    S3 = np.zeros((32, 16), np.float32)
    T1 = np.zeros((16, 32), np.float32)
    T2 = np.zeros((16, 32), np.float32)
    T3 = np.zeros((32, 16), np.float32)
    for i in range(3):
        for j in range(3):
            for k in range(3):
                m = i * 9 + j * 3 + k
                S1[j * 3 + i, m] = 1.0
                S2[j * 3 + k, m] = 1.0
                S3[m, i * 3 + k] = 1.0
                T1[i * 3 + j, m] = 1.0
                T2[j * 3 + k, m] = 1.0
                T3[m, i * 3 + k] = 1.0
    return S1, S2, S3, T1, T2, T3


_S1, _S2, _S3, _T1, _T2, _T3 = _sel_matrices()


def _k0_body(h_ref, wr_ref, wcb_ref, b1_ref, a_ref, b_ref):
    h = h_ref[...]
    a_ref[...] = jnp.dot(h, wr_ref[...], preferred_element_type=jnp.float32)
    b_ref[...] = jnp.dot(h, wcb_ref[...],
                         preferred_element_type=jnp.float32) + b1_ref[...]


def _k0(hp, wr, wc, b1):
    grid = NP // NBLK
    return pl.pallas_call(
        _k0_body,
        grid=(grid,),
        in_specs=[
            pl.BlockSpec((NBLK, H), lambda i: (i, 0)),
            pl.BlockSpec((H, H), lambda i: (0, 0)),
            pl.BlockSpec((H, H), lambda i: (0, 0)),
            pl.BlockSpec((1, H), lambda i: (0, 0)),
        ],
        out_specs=[
            pl.BlockSpec((NBLK, H), lambda i: (i, 0)),
            pl.BlockSpec((NBLK, H), lambda i: (i, 0)),
        ],
        out_shape=[
            jax.ShapeDtypeStruct((NP, H), jnp.float32),
            jax.ShapeDtypeStruct((NP, H), jnp.float32),
        ],
        interpret=_INTERPRET,
    )(hp, wr, wc, b1)


def _k1a_body(row3, col3, a_hbm, b_hbm,
              ag_o, bg_o,
              idxr, idxc, abuf, bbuf,
              gsem0, gsem1, wsem):
    wid = lax.axis_index("s") * 2 + lax.axis_index("c")
    base = wid * EPW
    pltpu.sync_copy(row3.at[wid], idxr)
    pltpu.sync_copy(col3.at[wid], idxc)

    def gather(j, b, sem):
        return [
            pltpu.async_copy(a_hbm.at[idxr.at[j]], abuf.at[b], sem),
            pltpu.async_copy(b_hbm.at[idxc.at[j]], bbuf.at[b], sem),
        ]

    def writeback(j, b):
        off = base + j * C
        return [
            pltpu.async_copy(abuf.at[b], ag_o.at[pl.ds(off, C)], wsem),
            pltpu.async_copy(bbuf.at[b], bg_o.at[pl.ds(off, C)], wsem),
        ]

    def pair(k, carry):
        j0 = 2 * k
        j1 = 2 * k + 1
        g0 = gather(j0, 0, gsem0)
        g1 = gather(j1, 1, gsem1)
        for cp in g0:
            cp.wait()
        w0 = writeback(j0, 0)
        for cp in g1:
            cp.wait()
        w1 = writeback(j1, 1)
        for cp in w0 + w1:
            cp.wait()
        return carry

    lax.fori_loop(0, NCH // 2, pair, 0)


def _k1a(row3, col3, A, B):
    f32 = jnp.float32
    mesh = plsc.VectorSubcoreMesh(core_axis_name="c", subcore_axis_name="s")
    fn = functools.partial(
        pl.kernel,
        out_type=[
            jax.ShapeDtypeStruct((EP2, H), f32),
            jax.ShapeDtypeStruct((EP2, H), f32),
        ],
        mesh=mesh,
        scratch_types=[
            pltpu.VMEM((NCH, C), jnp.int32),
            pltpu.VMEM((NCH, C), jnp.int32),
            pltpu.VMEM((2, C, H), f32),
            pltpu.VMEM((2, C, H), f32),
            pltpu.SemaphoreType.DMA,
            pltpu.SemaphoreType.DMA,
            pltpu.SemaphoreType.DMA,
        ],
    )(_k1a_body)
    return fn(row3, col3, A, B)


def _k1v_body(row3, col3, vec_hbm,
              zi_o, zj_o,
              idxr, idxc, zibuf, zjbuf,
              gsem0, gsem1, wsem):
    wid = lax.axis_index("s") * 2 + lax.axis_index("c")
    base = wid * EPW
    pltpu.sync_copy(row3.at[wid], idxr)
    pltpu.sync_copy(col3.at[wid], idxc)

    def gather(j, b, sem):
        return [
            pltpu.async_copy(vec_hbm.at[idxr.at[j]], zibuf.at[b], sem),
            pltpu.async_copy(vec_hbm.at[idxc.at[j]], zjbuf.at[b], sem),
        ]

    def writeback(j, b):
        off = base + j * C
        return [
            pltpu.async_copy(zibuf.at[b], zi_o.at[pl.ds(off, C)], wsem),
            pltpu.async_copy(zjbuf.at[b], zj_o.at[pl.ds(off, C)], wsem),
        ]

    def pair(k, carry):
        j0 = 2 * k
        j1 = 2 * k + 1
        g0 = gather(j0, 0, gsem0)
        g1 = gather(j1, 1, gsem1)
        for cp in g0:
            cp.wait()
        w0 = writeback(j0, 0)
        for cp in g1:
            cp.wait()
        w1 = writeback(j1, 1)
        for cp in w0 + w1:
            cp.wait()
        return carry

    lax.fori_loop(0, NCH // 2, pair, 0)


def _k1v(row3, col3, vecp):
    f32 = jnp.float32
    mesh = plsc.VectorSubcoreMesh(core_axis_name="c", subcore_axis_name="s")
    fn = functools.partial(
        pl.kernel,
        out_type=[
            jax.ShapeDtypeStruct((EP2, 16), f32),
            jax.ShapeDtypeStruct((EP2, 16), f32),
        ],
        mesh=mesh,
        scratch_types=[
            pltpu.VMEM((NCH, C), jnp.int32),
            pltpu.VMEM((NCH, C), jnp.int32),
            pltpu.VMEM((2, C, 16), f32),
            pltpu.VMEM((2, C, 16), f32),
            pltpu.SemaphoreType.DMA,
            pltpu.SemaphoreType.DMA,
            pltpu.SemaphoreType.DMA,
        ],
        compiler_params=pltpu.CompilerParams(use_tc_tiling_on_sc=False),
    )(_k1v_body)
    return fn(row3, col3, vecp)


NPS = NP // 16


def _make_k3(width, tiled):
    f32 = jnp.float32

    def body(row3, val_hbm, zeros_hbm, acc_o, idx, vbuf0, vbuf1, acc_sh,
             lsem0, lsem1):
        cid = lax.axis_index("c")
        sid = lax.axis_index("s")
        wid = sid * 2 + cid
        base = wid * EPW
        pltpu.sync_copy(row3.at[wid], idx)
        pltpu.sync_copy(zeros_hbm, acc_sh.at[pl.ds(sid * NPS, NPS)])
        plsc.subcore_barrier()

        def pair(k, carry):
            j0 = 2 * k
            j1 = 2 * k + 1
            cp0 = pltpu.async_copy(val_hbm.at[pl.ds(base + j0 * C, C)], vbuf0, lsem0)
            cp1 = pltpu.async_copy(val_hbm.at[pl.ds(base + j1 * C, C)], vbuf1, lsem1)
            cp0.wait()
            pltpu.sync_copy(vbuf0, acc_sh.at[idx.at[j0]], add=True)
            cp1.wait()
            pltpu.sync_copy(vbuf1, acc_sh.at[idx.at[j1]], add=True)
            return carry

        lax.fori_loop(0, NCH // 2, pair, 0)
        plsc.subcore_barrier()
        r = sid * NPS
        pltpu.sync_copy(acc_sh.at[pl.ds(r, NPS)], acc_o.at[cid, pl.ds(r, NPS)])

    params = None if tiled else pltpu.CompilerParams(use_tc_tiling_on_sc=False)
    mesh = plsc.VectorSubcoreMesh(core_axis_name="c", subcore_axis_name="s")
    fn = functools.partial(
        pl.kernel,
        out_type=jax.ShapeDtypeStruct((2, NP, width), f32),
        name=f"k3_{width}",
        mesh=mesh,
        scratch_types=[
            pltpu.VMEM((NCH, C), jnp.int32),
            pltpu.VMEM((C, width), f32),
            pltpu.VMEM((C, width), f32),
            pltpu.VMEM_SHARED((NP, width), f32),
            pltpu.SemaphoreType.DMA,
            pltpu.SemaphoreType.DMA,
        ],
        compiler_params=params,
    )(body)
    return fn


def _k2_body(ag_ref, bg_ref, zi_ref, zj_ref, fea_ref,
             wsf_ref, w2_ref, b2_ref, ow1_ref, ob1_ref, ow2_ref, ob2_ref,
             s1_ref, s2_ref, s3_ref, t1_ref, t2_ref, t3_ref, cnt_ref,
             msg_ref, tail_ref):
    zi = zi_ref[...]
    zj = zj_ref[...]
    t = (jnp.dot(zj, s1_ref[...], preferred_element_type=jnp.float32)
         * jnp.dot(zi, s2_ref[...], preferred_element_type=jnp.float32))
    scal = jnp.dot(t, s3_ref[...], preferred_element_type=jnp.float32)
    ss = jnp.sum(scal * scal, axis=1, keepdims=True)
    nrm = jnp.sqrt(ss)
    scal = scal * (1.0 / jnp.maximum(nrm, 1e-12))

    u = jnp.concatenate([scal, fea_ref[...]], axis=1)
    x = _silu(ag_ref[...] + bg_ref[...]
              + jnp.dot(u, wsf_ref[...], preferred_element_type=jnp.float32))
    msg = _silu(jnp.dot(x.astype(jnp.bfloat16), w2_ref[...],
                        preferred_element_type=jnp.float32) + b2_ref[...])

    y = _silu(jnp.dot(msg.astype(jnp.bfloat16), ow1_ref[...],
                      preferred_element_type=jnp.float32) + ob1_ref[...])
    vs = jnp.dot(y.astype(jnp.bfloat16), ow2_ref[...],
                 preferred_element_type=jnp.float32) + ob2_ref[...]

    t2 = (jnp.dot(zj, t1_ref[...], preferred_element_type=jnp.float32)
          * jnp.dot(vs, t2_ref[...], preferred_element_type=jnp.float32))
    vec = jnp.dot(t2, t3_ref[...], preferred_element_type=jnp.float32)

    msg_ref[...] = msg
    tail_ref[...] = vec + cnt_ref[...]


def _k2(half, ag, bg, zi, zj, fea, wsf, w2, b2, ow1, ob1, ow2, ob2):
    grid = EP2 // EBLK
    hoff = half * grid
    full = lambda i: (0, 0)
    blk = lambda i: (i, 0)
    fblk = lambda i: (i + hoff, 0)
    cnt = np.zeros((1, 16), np.float32)
    cnt[0, 9] = 1.0
    return pl.pallas_call(
        _k2_body,
        grid=(grid,),
        in_specs=[
            pl.BlockSpec((EBLK, H), blk),
            pl.BlockSpec((EBLK, H), blk),
            pl.BlockSpec((EBLK, 16), blk),
            pl.BlockSpec((EBLK, 16), blk),
            pl.BlockSpec((EBLK, 16), fblk),
            pl.BlockSpec((32, H), full),
            pl.BlockSpec((H, H), full),
            pl.BlockSpec((1, H), full),
            pl.BlockSpec((H, H), full),
            pl.BlockSpec((1, H), full),
            pl.BlockSpec((H, 16), full),
            pl.BlockSpec((1, 16), full),
            pl.BlockSpec((16, 32), full),
            pl.BlockSpec((16, 32), full),
            pl.BlockSpec((32, 16), full),
            pl.BlockSpec((16, 32), full),
            pl.BlockSpec((16, 32), full),
            pl.BlockSpec((32, 16), full),
            pl.BlockSpec((1, 16), full),
        ],
        out_specs=[
            pl.BlockSpec((EBLK, H), blk),
            pl.BlockSpec((EBLK, 16), blk),
        ],
        out_shape=[
            jax.ShapeDtypeStruct((EP2, H), jnp.float32),
            jax.ShapeDtypeStruct((EP2, 16), jnp.float32),
        ],
        interpret=_INTERPRET,
    )(ag, bg, zi, zj, fea, wsf, w2, b2, ow1, ob1, ow2, ob2,
      jnp.asarray(_S1), jnp.asarray(_S2), jnp.asarray(_S3),
      jnp.asarray(_T1), jnp.asarray(_T2), jnp.asarray(_T3),
      jnp.asarray(cnt))


def _k4_body(h_ref, accm0_ref, accm1_ref, accv0_ref, accv1_ref, vecp_ref,
             wn1a_ref, wn1b_ref, nb1_ref, wn2_ref, nb2_ref,
             hnew_ref, vout_ref):
    h = h_ref[...]
    tot = (accm0_ref[0] + accm0_ref[1]
           + accm1_ref[0] + accm1_ref[1])
    vsum = (accv0_ref[0] + accv0_ref[1]
            + accv1_ref[0] + accv1_ref[1])
    cnt = vsum[:, 9:10]
    recip = 1.0 / jnp.maximum(cnt, 1.0)
    vout_ref[...] = vsum * recip + vecp_ref[...]
    z = _silu(jnp.dot(h, wn1a_ref[...], preferred_element_type=jnp.float32)
              + jnp.dot(tot, wn1b_ref[...], preferred_element_type=jnp.float32)
              + nb1_ref[...])
    hnew_ref[...] = jnp.dot(z, wn2_ref[...], preferred_element_type=jnp.float32) + nb2_ref[...] + h


def _k4(hp, accm0, accm1, accv0, accv1, vecp, wn1a, wn1b, nb1, wn2, nb2):
    grid = NP // NBLK
    full = lambda i: (0, 0)
    return pl.pallas_call(
        _k4_body,
        grid=(grid,),
        in_specs=[
            pl.BlockSpec((NBLK, H), lambda i: (i, 0)),
            pl.BlockSpec((2, NBLK, H), lambda i: (0, i, 0)),
            pl.BlockSpec((2, NBLK, H), lambda i: (0, i, 0)),
            pl.BlockSpec((2, NBLK, 16), lambda i: (0, i, 0)),
            pl.BlockSpec((2, NBLK, 16), lambda i: (0, i, 0)),
            pl.BlockSpec((NBLK, 16), lambda i: (i, 0)),
            pl.BlockSpec((H, H), full),
            pl.BlockSpec((H, H), full),
            pl.BlockSpec((1, H), full),
            pl.BlockSpec((H, H), full),
            pl.BlockSpec((1, H), full),
        ],
        out_specs=[
            pl.BlockSpec((NBLK, H), lambda i: (i, 0)),
            pl.BlockSpec((NBLK, 16), lambda i: (i, 0)),
        ],
        out_shape=[
            jax.ShapeDtypeStruct((NP, H), jnp.float32),
            jax.ShapeDtypeStruct((NP, 16), jnp.float32),
        ],
        interpret=_INTERPRET,
    )(hp, accm0, accm1, accv0, accv1, vecp, wn1a, wn1b, nb1, wn2, nb2)


def kernel(vectors, h, edge_index, edge_fea,
           in_w1, in_b1, in_w2, in_b2,
           out_w1, out_b1, out_w2, out_b2,
           node_w1, node_b1, node_w2, node_b2):
    f32 = jnp.float32
    row = edge_index[0].astype(jnp.int32)
    col = edge_index[1].astype(jnp.int32)
    rowp = jnp.pad(row, (0, EP - E), constant_values=NP - 1)
    colp = jnp.pad(col, (0, EP - E))
    feap = jnp.pad(edge_fea.astype(f32), ((0, EP - E), (0, 16 - 4)))

    hp = jnp.pad(h.astype(f32), ((0, NP - N), (0, 0)))
    vecp = jnp.pad(vectors.astype(f32).reshape(N, V * V), ((0, NP - N), (0, 16 - V * V)))

    wr = in_w1[9:137]
    wc = in_w1[137:265]
    wsf = jnp.zeros((32, H), f32).at[0:9].set(in_w1[0:9]).at[16:20].set(in_w1[265:269])
    b1 = in_b1.reshape(1, H)

    A, B = _k0(hp, wr, wc, b1)

    row4 = rowp.reshape(2, NW, NCH, C)
    col4 = colp.reshape(2, NW, NCH, C)

    ow2p = jnp.zeros((H, 16), f32).at[:, :9].set(out_w2)
    ob2p = jnp.zeros((1, 16), f32).at[0, :9].set(out_b2)
    w2b = in_w2.astype(jnp.bfloat16)
    ow1b = out_w1.astype(jnp.bfloat16)
    ow2b = ow2p.astype(jnp.bfloat16)
    zm = jnp.zeros((NPS, H), f32)
    zv = jnp.zeros((NPS, 16), f32)
    k3m = _make_k3(H, True)
    k3v = _make_k3(16, False)

    accms = []
    accvs = []
    for hlf in range(2):
        ag, bg = _k1a(row4[hlf], col4[hlf], A, B)
        zi, zj = _k1v(row4[hlf], col4[hlf], vecp)
        msg, tail = _k2(hlf, ag, bg, zi, zj, feap,
                        wsf, w2b, in_b2.reshape(1, H),
                        ow1b, out_b1.reshape(1, H), ow2b, ob2p)
        accms.append(k3m(row4[hlf], msg, zm))
        accvs.append(k3v(row4[hlf], tail, zv))

    wn1a = node_w1[:H]
    wn1b = node_w1[H:]

    hnew, vout = _k4(hp, accms[0], accms[1], accvs[0], accvs[1], vecp,
                     wn1a, wn1b, node_b1.reshape(1, H),
                     node_w2, node_b2.reshape(1, H))

    return (vout[:N, :9].reshape(N, 3, V), hnew[:N])


# speedup vs baseline: 4.6260x; 1.0002x over previous
import functools
import numpy as np
import jax
import jax.numpy as jnp
from jax import lax
from jax.experimental import pallas as pl
from jax.experimental.pallas import tpu as pltpu
from jax.experimental.pallas import tpu_sc as plsc

N = 10000
E = 320000
V = 3
H = 128

NP = 10240
NW = 32
C = 128
EP = 327680
EP2 = EP // 2
EPW = EP2 // NW
NCH = EPW // C
EBLK = 2048
NBLK = 512

def _silu(x):
    return x * jax.nn.sigmoid(x)


def _sel_matrices():
    S1 = np.zeros((16, 32), np.float32)
    S2 = np.zeros((16, 32), np.float32)
    S3 = np.zeros((32, 16), np.float32)
    T1 = np.zeros((16, 32), np.float32)
    T2 = np.zeros((16, 32), np.float32)
    T3 = np.zeros((32, 16), np.float32)
    for i in range(3):
        for j in range(3):
            for k in range(3):
                m = i * 9 + j * 3 + k
                S1[j * 3 + i, m] = 1.0
                S2[j * 3 + k, m] = 1.0
                S3[m, i * 3 + k] = 1.0
                T1[i * 3 + j, m] = 1.0
                T2[j * 3 + k, m] = 1.0
                T3[m, i * 3 + k] = 1.0
    return S1, S2, S3, T1, T2, T3


_S1, _S2, _S3, _T1, _T2, _T3 = _sel_matrices()


def _k0_body(h_ref, wr_ref, wcb_ref, b1_ref, a_ref, b_ref):
    h = h_ref[...]
    a_ref[...] = jnp.dot(h, wr_ref[...], preferred_element_type=jnp.float32)
    b_ref[...] = jnp.dot(h, wcb_ref[...],
                         preferred_element_type=jnp.float32) + b1_ref[...]


def _k0(hp, wr, wc, b1):
    grid = NP // NBLK
    return pl.pallas_call(
        _k0_body,
        grid=(grid,),
        in_specs=[
            pl.BlockSpec((NBLK, H), lambda i: (i, 0)),
            pl.BlockSpec((H, H), lambda i: (0, 0)),
            pl.BlockSpec((H, H), lambda i: (0, 0)),
            pl.BlockSpec((1, H), lambda i: (0, 0)),
        ],
        out_specs=[
            pl.BlockSpec((NBLK, H), lambda i: (i, 0)),
            pl.BlockSpec((NBLK, H), lambda i: (i, 0)),
        ],
        out_shape=[
            jax.ShapeDtypeStruct((NP, H), jnp.float32),
            jax.ShapeDtypeStruct((NP, H), jnp.float32),
        ],
    )(hp, wr, wc, b1)


def _k1a_body(row3, col3, a_hbm, b_hbm,
              ag_o, bg_o,
              idxr, idxc, abuf, bbuf,
              gsem0, gsem1, wsem):
    wid = lax.axis_index("s") * 2 + lax.axis_index("c")
    base = wid * EPW
    pltpu.sync_copy(row3.at[wid], idxr)
    pltpu.sync_copy(col3.at[wid], idxc)

    def gather(j, b, sem):
        return [
            pltpu.async_copy(a_hbm.at[idxr.at[j]], abuf.at[b], sem),
            pltpu.async_copy(b_hbm.at[idxc.at[j]], bbuf.at[b], sem),
        ]

    def writeback(j, b):
        off = base + j * C
        return [
            pltpu.async_copy(abuf.at[b], ag_o.at[pl.ds(off, C)], wsem),
            pltpu.async_copy(bbuf.at[b], bg_o.at[pl.ds(off, C)], wsem),
        ]

    def pair(k, carry):
        j0 = 2 * k
        j1 = 2 * k + 1
        g0 = gather(j0, 0, gsem0)
        g1 = gather(j1, 1, gsem1)
        for cp in g0:
            cp.wait()
        w0 = writeback(j0, 0)
        for cp in g1:
            cp.wait()
        w1 = writeback(j1, 1)
        for cp in w0 + w1:
            cp.wait()
        return carry

    lax.fori_loop(0, NCH // 2, pair, 0)


def _k1a(row3, col3, A, B):
    f32 = jnp.float32
    mesh = plsc.VectorSubcoreMesh(core_axis_name="c", subcore_axis_name="s")
    fn = functools.partial(
        pl.kernel,
        out_type=[
            jax.ShapeDtypeStruct((EP2, H), f32),
            jax.ShapeDtypeStruct((EP2, H), f32),
        ],
        mesh=mesh,
        scratch_types=[
            pltpu.VMEM((NCH, C), jnp.int32),
            pltpu.VMEM((NCH, C), jnp.int32),
            pltpu.VMEM((2, C, H), f32),
            pltpu.VMEM((2, C, H), f32),
            pltpu.SemaphoreType.DMA,
            pltpu.SemaphoreType.DMA,
            pltpu.SemaphoreType.DMA,
        ],
    )(_k1a_body)
    return fn(row3, col3, A, B)


def _k1v_body(row3, col3, vec_hbm,
              zi_o, zj_o,
              idxr, idxc, zibuf, zjbuf,
              gsem0, gsem1, wsem):
    wid = lax.axis_index("s") * 2 + lax.axis_index("c")
    base = wid * EPW
    pltpu.sync_copy(row3.at[wid], idxr)
    pltpu.sync_copy(col3.at[wid], idxc)

    def gather(j, b, sem):
        return [
            pltpu.async_copy(vec_hbm.at[idxr.at[j]], zibuf.at[b], sem),
            pltpu.async_copy(vec_hbm.at[idxc.at[j]], zjbuf.at[b], sem),
        ]

    def writeback(j, b):
        off = base + j * C
        return [
            pltpu.async_copy(zibuf.at[b], zi_o.at[pl.ds(off, C)], wsem),
            pltpu.async_copy(zjbuf.at[b], zj_o.at[pl.ds(off, C)], wsem),
        ]

    def pair(k, carry):
        j0 = 2 * k
        j1 = 2 * k + 1
        g0 = gather(j0, 0, gsem0)
        g1 = gather(j1, 1, gsem1)
        for cp in g0:
            cp.wait()
        w0 = writeback(j0, 0)
        for cp in g1:
            cp.wait()
        w1 = writeback(j1, 1)
        for cp in w0 + w1:
            cp.wait()
        return carry

    lax.fori_loop(0, NCH // 2, pair, 0)


def _k1v(row3, col3, vecp):
    f32 = jnp.float32
    mesh = plsc.VectorSubcoreMesh(core_axis_name="c", subcore_axis_name="s")
    fn = functools.partial(
        pl.kernel,
        out_type=[
            jax.ShapeDtypeStruct((EP2, 16), f32),
            jax.ShapeDtypeStruct((EP2, 16), f32),
        ],
        mesh=mesh,
        scratch_types=[
            pltpu.VMEM((NCH, C), jnp.int32),
            pltpu.VMEM((NCH, C), jnp.int32),
            pltpu.VMEM((2, C, 16), f32),
            pltpu.VMEM((2, C, 16), f32),
            pltpu.SemaphoreType.DMA,
            pltpu.SemaphoreType.DMA,
            pltpu.SemaphoreType.DMA,
        ],
        compiler_params=pltpu.CompilerParams(use_tc_tiling_on_sc=False),
    )(_k1v_body)
    return fn(row3, col3, vecp)


NPS = NP // 16


def _make_k3(width, tiled):
    f32 = jnp.float32

    def body(row3, val_hbm, zeros_hbm, acc_o, idx, vbuf0, vbuf1, acc_sh,
             lsem0, lsem1):
        cid = lax.axis_index("c")
        sid = lax.axis_index("s")
        wid = sid * 2 + cid
        base = wid * EPW
        pltpu.sync_copy(row3.at[wid], idx)
        pltpu.sync_copy(zeros_hbm, acc_sh.at[pl.ds(sid * NPS, NPS)])
        plsc.subcore_barrier()

        def pair(k, carry):
            j0 = 2 * k
            j1 = 2 * k + 1
            cp0 = pltpu.async_copy(val_hbm.at[pl.ds(base + j0 * C, C)], vbuf0, lsem0)
            cp1 = pltpu.async_copy(val_hbm.at[pl.ds(base + j1 * C, C)], vbuf1, lsem1)
            cp0.wait()
            pltpu.sync_copy(vbuf0, acc_sh.at[idx.at[j0]], add=True)
            cp1.wait()
            pltpu.sync_copy(vbuf1, acc_sh.at[idx.at[j1]], add=True)
            return carry

        lax.fori_loop(0, NCH // 2, pair, 0)
        plsc.subcore_barrier()
        r = sid * NPS
        pltpu.sync_copy(acc_sh.at[pl.ds(r, NPS)], acc_o.at[cid, pl.ds(r, NPS)])

    params = None if tiled else pltpu.CompilerParams(use_tc_tiling_on_sc=False)
    mesh = plsc.VectorSubcoreMesh(core_axis_name="c", subcore_axis_name="s")
    fn = functools.partial(
        pl.kernel,
        out_type=jax.ShapeDtypeStruct((2, NP, width), f32),
        name=f"k3_{width}",
        mesh=mesh,
        scratch_types=[
            pltpu.VMEM((NCH, C), jnp.int32),
            pltpu.VMEM((C, width), f32),
            pltpu.VMEM((C, width), f32),
            pltpu.VMEM_SHARED((NP, width), f32),
            pltpu.SemaphoreType.DMA,
            pltpu.SemaphoreType.DMA,
        ],
        compiler_params=params,
    )(body)
    return fn


def _k2_body(ag_ref, bg_ref, zi_ref, zj_ref, fea_ref,
             wsf_ref, w2_ref, b2_ref, ow1_ref, ob1_ref, ow2_ref, ob2_ref,
             s1_ref, s2_ref, s3_ref, t1_ref, t2_ref, t3_ref, cnt_ref,
             msg_ref, tail_ref):
    zi = zi_ref[...]
    zj = zj_ref[...]
    t = (jnp.dot(zj, s1_ref[...], preferred_element_type=jnp.float32)
         * jnp.dot(zi, s2_ref[...], preferred_element_type=jnp.float32))
    scal = jnp.dot(t, s3_ref[...], preferred_element_type=jnp.float32)
    ss = jnp.sum(scal * scal, axis=1, keepdims=True)
    nrm = jnp.sqrt(ss)
    scal = scal * (1.0 / jnp.maximum(nrm, 1e-12))

    u = jnp.concatenate([scal, fea_ref[...]], axis=1)
    x = _silu(ag_ref[...] + bg_ref[...]
              + jnp.dot(u, wsf_ref[...], preferred_element_type=jnp.float32))
    msg = _silu(jnp.dot(x.astype(jnp.bfloat16), w2_ref[...],
                        preferred_element_type=jnp.float32) + b2_ref[...])

    y = _silu(jnp.dot(msg.astype(jnp.bfloat16), ow1_ref[...],
                      preferred_element_type=jnp.float32) + ob1_ref[...])
    vs = jnp.dot(y.astype(jnp.bfloat16), ow2_ref[...],
                 preferred_element_type=jnp.float32) + ob2_ref[...]

    t2 = (jnp.dot(zj, t1_ref[...], preferred_element_type=jnp.float32)
          * jnp.dot(vs, t2_ref[...], preferred_element_type=jnp.float32))
    vec = jnp.dot(t2, t3_ref[...], preferred_element_type=jnp.float32)

    msg_ref[...] = msg
    tail_ref[...] = vec + cnt_ref[...]


def _k2(half, ag, bg, zi, zj, fea, wsf, w2, b2, ow1, ob1, ow2, ob2):
    grid = EP2 // EBLK
    hoff = half * grid
    full = lambda i: (0, 0)
    blk = lambda i: (i, 0)
    fblk = lambda i: (i + hoff, 0)
    cnt = np.zeros((1, 16), np.float32)
    cnt[0, 9] = 1.0
    return pl.pallas_call(
        _k2_body,
        grid=(grid,),
        in_specs=[
            pl.BlockSpec((EBLK, H), blk),
            pl.BlockSpec((EBLK, H), blk),
            pl.BlockSpec((EBLK, 16), blk),
            pl.BlockSpec((EBLK, 16), blk),
            pl.BlockSpec((EBLK, 16), fblk),
            pl.BlockSpec((32, H), full),
            pl.BlockSpec((H, H), full),
            pl.BlockSpec((1, H), full),
            pl.BlockSpec((H, H), full),
            pl.BlockSpec((1, H), full),
            pl.BlockSpec((H, 16), full),
            pl.BlockSpec((1, 16), full),
            pl.BlockSpec((16, 32), full),
            pl.BlockSpec((16, 32), full),
            pl.BlockSpec((32, 16), full),
            pl.BlockSpec((16, 32), full),
            pl.BlockSpec((16, 32), full),
            pl.BlockSpec((32, 16), full),
            pl.BlockSpec((1, 16), full),
        ],
        out_specs=[
            pl.BlockSpec((EBLK, H), blk),
            pl.BlockSpec((EBLK, 16), blk),
        ],
        out_shape=[
            jax.ShapeDtypeStruct((EP2, H), jnp.float32),
            jax.ShapeDtypeStruct((EP2, 16), jnp.float32),
        ],
    )(ag, bg, zi, zj, fea, wsf, w2, b2, ow1, ob1, ow2, ob2,
      jnp.asarray(_S1), jnp.asarray(_S2), jnp.asarray(_S3),
      jnp.asarray(_T1), jnp.asarray(_T2), jnp.asarray(_T3),
      jnp.asarray(cnt))


def _k4_body(h_ref, accm0_ref, accm1_ref, accv0_ref, accv1_ref, vecp_ref,
             wn1a_ref, wn1b_ref, nb1_ref, wn2_ref, nb2_ref,
             hnew_ref, vout_ref):
    h = h_ref[...]
    tot = (accm0_ref[0] + accm0_ref[1]
           + accm1_ref[0] + accm1_ref[1])
    vsum = (accv0_ref[0] + accv0_ref[1]
            + accv1_ref[0] + accv1_ref[1])
    cnt = vsum[:, 9:10]
    recip = 1.0 / jnp.maximum(cnt, 1.0)
    vout_ref[...] = vsum * recip + vecp_ref[...]
    z = _silu(jnp.dot(h, wn1a_ref[...], preferred_element_type=jnp.float32)
              + jnp.dot(tot, wn1b_ref[...], preferred_element_type=jnp.float32)
              + nb1_ref[...])
    hnew_ref[...] = jnp.dot(z, wn2_ref[...], preferred_element_type=jnp.float32) + nb2_ref[...] + h


def _k4(hp, accm0, accm1, accv0, accv1, vecp, wn1a, wn1b, nb1, wn2, nb2):
    grid = NP // NBLK
    full = lambda i: (0, 0)
    return pl.pallas_call(
        _k4_body,
        grid=(grid,),
        in_specs=[
            pl.BlockSpec((NBLK, H), lambda i: (i, 0)),
            pl.BlockSpec((2, NBLK, H), lambda i: (0, i, 0)),
            pl.BlockSpec((2, NBLK, H), lambda i: (0, i, 0)),
            pl.BlockSpec((2, NBLK, 16), lambda i: (0, i, 0)),
            pl.BlockSpec((2, NBLK, 16), lambda i: (0, i, 0)),
            pl.BlockSpec((NBLK, 16), lambda i: (i, 0)),
            pl.BlockSpec((H, H), full),
            pl.BlockSpec((H, H), full),
            pl.BlockSpec((1, H), full),
            pl.BlockSpec((H, H), full),
            pl.BlockSpec((1, H), full),
        ],
        out_specs=[
            pl.BlockSpec((NBLK, H), lambda i: (i, 0)),
            pl.BlockSpec((NBLK, 16), lambda i: (i, 0)),
        ],
        out_shape=[
            jax.ShapeDtypeStruct((NP, H), jnp.float32),
            jax.ShapeDtypeStruct((NP, 16), jnp.float32),
        ],
    )(hp, accm0, accm1, accv0, accv1, vecp, wn1a, wn1b, nb1, wn2, nb2)


def kernel(vectors, h, edge_index, edge_fea,
           in_w1, in_b1, in_w2, in_b2,
           out_w1, out_b1, out_w2, out_b2,
           node_w1, node_b1, node_w2, node_b2):
    f32 = jnp.float32
    row = edge_index[0].astype(jnp.int32)
    col = edge_index[1].astype(jnp.int32)
    rowp = jnp.pad(row, (0, EP - E), constant_values=NP - 1)
    colp = jnp.pad(col, (0, EP - E))
    feap = jnp.pad(edge_fea.astype(f32), ((0, EP - E), (0, 16 - 4)))

    hp = jnp.pad(h.astype(f32), ((0, NP - N), (0, 0)))
    vecp = jnp.pad(vectors.astype(f32).reshape(N, V * V), ((0, NP - N), (0, 16 - V * V)))

    wr = in_w1[9:137]
    wc = in_w1[137:265]
    wsf = jnp.zeros((32, H), f32).at[0:9].set(in_w1[0:9]).at[16:20].set(in_w1[265:269])
    b1 = in_b1.reshape(1, H)

    A, B = _k0(hp, wr, wc, b1)

    row4 = rowp.reshape(2, NW, NCH, C)
    col4 = colp.reshape(2, NW, NCH, C)

    ow2p = jnp.zeros((H, 16), f32).at[:, :9].set(out_w2)
    ob2p = jnp.zeros((1, 16), f32).at[0, :9].set(out_b2)
    w2b = in_w2.astype(jnp.bfloat16)
    ow1b = out_w1.astype(jnp.bfloat16)
    ow2b = ow2p.astype(jnp.bfloat16)
    zm = jnp.zeros((NPS, H), f32)
    zv = jnp.zeros((NPS, 16), f32)
    k3m = _make_k3(H, True)
    k3v = _make_k3(16, False)

    accms = []
    accvs = []
    for hlf in range(2):
        ag, bg = _k1a(row4[hlf], col4[hlf], A, B)
        zi, zj = _k1v(row4[hlf], col4[hlf], vecp)
        msg, tail = _k2(hlf, ag, bg, zi, zj, feap,
                        wsf, w2b, in_b2.reshape(1, H),
                        ow1b, out_b1.reshape(1, H), ow2b, ob2p)
        accms.append(k3m(row4[hlf], msg, zm))
        accvs.append(k3v(row4[hlf], tail, zv))

    wn1a = node_w1[:H]
    wn1b = node_w1[H:]

    hnew, vout = _k4(hp, accms[0], accms[1], accvs[0], accvs[1], vecp,
                     wn1a, wn1b, node_b1.reshape(1, H),
                     node_w2, node_b2.reshape(1, H))

    return (vout[:N, :9].reshape(N, 3, V), hnew[:N])
